# Initial kernel scaffold; baseline (speedup 1.0000x reference)
import functools

import jax
import jax.numpy as jnp
from jax import lax
from jax.experimental import pallas as pl
from jax.experimental.pallas import tpu as pltpu
from jax.experimental.pallas import tpu_sc as plsc

N = 10000
E = 320000
D = 128
C = 60

NC = 2
NS = 16
NW = NC * NS
PER_W = E // NW
CHUNK = 400
NSTEPS = PER_W // CHUNK
NPAD = 10240
RPT = NPAD // NW
ROWS_PER_TILE = NPAD // NS

_SC_MESH = plsc.VectorSubcoreMesh(core_axis_name="c", subcore_axis_name="s")


@functools.partial(
    pl.kernel,
    out_type=jax.ShapeDtypeStruct((2, 2, NPAD, 16), jnp.float32),
    mesh=_SC_MESH,
    scratch_types=[
        pltpu.VMEM_SHARED((NPAD, 16), jnp.float32),
        pltpu.VMEM_SHARED((NPAD, 16), jnp.float32),
        pltpu.VMEM((CHUNK,), jnp.int32),
        pltpu.VMEM((CHUNK,), jnp.int32),
        pltpu.VMEM((CHUNK, 16), jnp.float32),
        pltpu.VMEM((ROWS_PER_TILE, 16), jnp.float32),
    ],
)
def _sc_degrees(src_hbm, dst_hbm, ones_hbm, zeros_hbm, out_hbm,
                deg_s, deg_d, sbuf, dbuf, obuf, zbuf):
    c = lax.axis_index("c")
    s = lax.axis_index("s")
    wid = c * NS + s

    pltpu.sync_copy(zeros_hbm, zbuf)
    r0 = s * ROWS_PER_TILE
    pltpu.sync_copy(zbuf, deg_s.at[pl.ds(r0, ROWS_PER_TILE)])
    pltpu.sync_copy(zbuf, deg_d.at[pl.ds(r0, ROWS_PER_TILE)])
    pltpu.sync_copy(ones_hbm, obuf)
    plsc.subcore_barrier()

    def step(j, carry):
        off = pl.multiple_of(j * CHUNK, 8)
        pltpu.sync_copy(src_hbm.at[wid, pl.ds(off, CHUNK)], sbuf)
        pltpu.sync_copy(dst_hbm.at[wid, pl.ds(off, CHUNK)], dbuf)
        pltpu.sync_copy(obuf, deg_s.at[sbuf], add=True)
        pltpu.sync_copy(obuf, deg_d.at[dbuf], add=True)
        return carry

    lax.fori_loop(0, NSTEPS, step, 0)
    plsc.subcore_barrier()

    pltpu.sync_copy(deg_s.at[pl.ds(r0, ROWS_PER_TILE)], zbuf)
    pltpu.sync_copy(zbuf, out_hbm.at[c, 0, pl.ds(r0, ROWS_PER_TILE)])
    pltpu.sync_copy(deg_d.at[pl.ds(r0, ROWS_PER_TILE)], zbuf)
    pltpu.sync_copy(zbuf, out_hbm.at[c, 1, pl.ds(r0, ROWS_PER_TILE)])


@functools.partial(
    pl.kernel,
    out_type=jax.ShapeDtypeStruct((2, NPAD, D), jnp.float32),
    mesh=_SC_MESH,
    scratch_types=[
        pltpu.VMEM_SHARED((NPAD, D), jnp.float32),
        pltpu.VMEM((CHUNK,), jnp.int32),
        pltpu.VMEM((CHUNK,), jnp.int32),
        pltpu.VMEM((CHUNK, D), jnp.float32),
        pltpu.SemaphoreType.DMA,
    ],
)
def _sc_aggregate(feat_hbm, src_hbm, dst_hbm, zeros_hbm, out_hbm,
                  agg, sbuf, dbuf, rows, sem):
    c = lax.axis_index("c")
    s = lax.axis_index("s")
    wid = c * NS + s

    pltpu.sync_copy(zeros_hbm, rows.at[pl.ds(0, RPT)])
    pltpu.sync_copy(rows.at[pl.ds(0, RPT)], agg.at[pl.ds(s * 2 * RPT, RPT)])
    pltpu.sync_copy(rows.at[pl.ds(0, RPT)],
                    agg.at[pl.ds(s * 2 * RPT + RPT, RPT)])
    plsc.subcore_barrier()

    def step(j, carry):
        off = pl.multiple_of(j * CHUNK, 8)
        pltpu.sync_copy(src_hbm.at[wid, pl.ds(off, CHUNK)], sbuf)
        pltpu.sync_copy(dst_hbm.at[wid, pl.ds(off, CHUNK)], dbuf)
        pltpu.async_copy(feat_hbm.at[sbuf], rows, sem).wait()
        pltpu.sync_copy(rows, agg.at[dbuf], add=True)
        return carry

    lax.fori_loop(0, NSTEPS, step, 0)
    plsc.subcore_barrier()

    for t in range(2):
        r0 = s * 2 * RPT + t * RPT
        pltpu.sync_copy(agg.at[pl.ds(r0, RPT)], rows.at[pl.ds(0, RPT)])
        pltpu.sync_copy(rows.at[pl.ds(0, RPT)], out_hbm.at[c, pl.ds(r0, RPT)])


BLK = 1000
EPS = 1e-5


def _norm_from_deg(degp_blk, which):
    deg = degp_blk[0, which, :, 0] + degp_blk[1, which, :, 0]
    return lax.rsqrt(jnp.maximum(deg, 1.0))


def _embed_body(nf, w, b, degp, x_out, feat_out):
    x = jnp.dot(nf[...], w[...], preferred_element_type=jnp.float32) + b[...]
    x_out[...] = x
    ns = _norm_from_deg(degp[...], 0)
    feat_out[...] = x * ns[:, None]


def _tc_embed(nf, w_emb, b_emb, degp):
    return pl.pallas_call(
        _embed_body,
        grid=(N // BLK,),
        in_specs=[
            pl.BlockSpec((BLK, D), lambda i: (i, 0)),
            pl.BlockSpec((D, D), lambda i: (0, 0)),
            pl.BlockSpec((1, D), lambda i: (0, 0)),
            pl.BlockSpec((2, 2, BLK, 16), lambda i: (0, 0, i, 0)),
        ],
        out_specs=[
            pl.BlockSpec((BLK, D), lambda i: (i, 0)),
            pl.BlockSpec((BLK, D), lambda i: (i, 0)),
        ],
        out_shape=[
            jax.ShapeDtypeStruct((N, D), jnp.float32),
            jax.ShapeDtypeStruct((N, D), jnp.float32),
        ],
    )(nf, w_emb, b_emb, degp)


def _post_a_body(aggp, x, degp, w, b, h_out, st_out, acc):
    i = pl.program_id(0)
    nd = _norm_from_deg(degp[...], 1)
    a = (aggp[0] + aggp[1]) * nd[:, None]
    h = x[...] + jnp.dot(a, w[...], preferred_element_type=jnp.float32) + b[...]
    h_out[...] = h

    @pl.when(i == 0)
    def _():
        acc[...] = jnp.zeros_like(acc)

    acc[0, :] += jnp.sum(h, axis=0)
    acc[1, :] += jnp.sum(h * h, axis=0)

    @pl.when(i == pl.num_programs(0) - 1)
    def _():
        st_out[...] = acc[...]


def _tc_post_a(aggp, x, degp, w, b):
    return pl.pallas_call(
        _post_a_body,
        grid=(N // BLK,),
        in_specs=[
            pl.BlockSpec((2, BLK, D), lambda i: (0, i, 0)),
            pl.BlockSpec((BLK, D), lambda i: (i, 0)),
            pl.BlockSpec((2, 2, BLK, 16), lambda i: (0, 0, i, 0)),
            pl.BlockSpec((D, D), lambda i: (0, 0)),
            pl.BlockSpec((1, D), lambda i: (0, 0)),
        ],
        out_specs=[
            pl.BlockSpec((BLK, D), lambda i: (i, 0)),
            pl.BlockSpec((2, D), lambda i: (0, 0)),
        ],
        out_shape=[
            jax.ShapeDtypeStruct((N, D), jnp.float32),
            jax.ShapeDtypeStruct((2, D), jnp.float32),
        ],
        scratch_shapes=[pltpu.VMEM((2, D), jnp.float32)],
    )(aggp, x, degp, w, b)


def _bn_relu(h, st, g, be):
    mean = st[0, :] / N
    var = st[1, :] / N - mean * mean
    inv = lax.rsqrt(var + EPS)
    hn = (h - mean) * inv * g + be
    return jnp.maximum(hn, 0.0)


def _post_b_body(h1, st, g, be, w, b, degp, feat_out):
    r = _bn_relu(h1[...], st[...], g[...], be[...])
    t = jnp.dot(r, w[...], preferred_element_type=jnp.float32) + b[...]
    ns = _norm_from_deg(degp[...], 0)
    feat_out[...] = t * ns[:, None]


def _tc_post_b(h1, st, g, be, w, b, degp):
    return pl.pallas_call(
        _post_b_body,
        grid=(N // BLK,),
        in_specs=[
            pl.BlockSpec((BLK, D), lambda i: (i, 0)),
            pl.BlockSpec((2, D), lambda i: (0, 0)),
            pl.BlockSpec((1, D), lambda i: (0, 0)),
            pl.BlockSpec((1, D), lambda i: (0, 0)),
            pl.BlockSpec((D, D), lambda i: (0, 0)),
            pl.BlockSpec((1, D), lambda i: (0, 0)),
            pl.BlockSpec((2, 2, BLK, 16), lambda i: (0, 0, i, 0)),
        ],
        out_specs=pl.BlockSpec((BLK, D), lambda i: (i, 0)),
        out_shape=jax.ShapeDtypeStruct((N, D), jnp.float32),
    )(h1, st, g, be, w, b, degp)


def _final_body(h2, st, g, be, w, b, w_cls, b_cls, h_out, label_out, acc):
    i = pl.program_id(0)
    r = _bn_relu(h2[...], st[...], g[...], be[...])
    ho = jnp.dot(r, w[...], preferred_element_type=jnp.float32) + b[...]
    h_out[...] = ho

    @pl.when(i == 0)
    def _():
        acc[...] = jnp.zeros_like(acc)

    acc[0, :] += jnp.sum(ho, axis=0)

    @pl.when(i == pl.num_programs(0) - 1)
    def _():
        y = acc[...] / N
        label_out[...] = (
            jnp.dot(y, w_cls[...], preferred_element_type=jnp.float32)
            + b_cls[...]
        )


def _tc_final(h2, st, g, be, w, b, w_cls, b_cls):
    return pl.pallas_call(
        _final_body,
        grid=(N // BLK,),
        in_specs=[
            pl.BlockSpec((BLK, D), lambda i: (i, 0)),
            pl.BlockSpec((2, D), lambda i: (0, 0)),
            pl.BlockSpec((1, D), lambda i: (0, 0)),
            pl.BlockSpec((1, D), lambda i: (0, 0)),
            pl.BlockSpec((D, D), lambda i: (0, 0)),
            pl.BlockSpec((1, D), lambda i: (0, 0)),
            pl.BlockSpec((D, C), lambda i: (0, 0)),
            pl.BlockSpec((1, C), lambda i: (0, 0)),
        ],
        out_specs=[
            pl.BlockSpec((BLK, D), lambda i: (i, 0)),
            pl.BlockSpec((1, C), lambda i: (0, 0)),
        ],
        out_shape=[
            jax.ShapeDtypeStruct((N, D), jnp.float32),
            jax.ShapeDtypeStruct((1, C), jnp.float32),
        ],
        scratch_shapes=[pltpu.VMEM((1, D), jnp.float32)],
    )(h2, st, g, be, w, b, w_cls, b_cls)


def kernel(node_features, edge_index, W_emb, b_emb, W_c1, b_c1, g1, be1,
           W_l2, b_l2, W_c2, b_c2, g2, be2, W_l3, b_l3, W_cls, b_cls):
    src2d = edge_index[0].reshape(NW, PER_W)
    dst2d = edge_index[1].reshape(NW, PER_W)
    ones16 = jnp.ones((CHUNK, 16), jnp.float32)
    zeros16 = jnp.zeros((ROWS_PER_TILE, 16), jnp.float32)
    zeros128 = jnp.zeros((RPT, D), jnp.float32)

    degp = _sc_degrees(src2d, dst2d, ones16, zeros16)
    x, feat1 = _tc_embed(node_features, W_emb, b_emb.reshape(1, D), degp)
    agg1 = _sc_aggregate(feat1, src2d, dst2d, zeros128)
    h1, st1 = _tc_post_a(agg1, x, degp, W_c1, b_c1.reshape(1, D))
    feat2 = _tc_post_b(h1, st1, g1.reshape(1, D), be1.reshape(1, D),
                       W_l2, b_l2.reshape(1, D), degp)
    agg2 = _sc_aggregate(feat2, src2d, dst2d, zeros128)
    h2, st2 = _tc_post_a(agg2, x, degp, W_c2, b_c2.reshape(1, D))
    h_out, label = _tc_final(h2, st2, g2.reshape(1, D), be2.reshape(1, D),
                             W_l3, b_l3.reshape(1, D), W_cls,
                             b_cls.reshape(1, C))
    return (h_out, label)


# speedup vs baseline: 6.8051x; 6.8051x over previous
import functools

import jax
import jax.numpy as jnp
from jax import lax
from jax.experimental import pallas as pl
from jax.experimental.pallas import tpu as pltpu
from jax.experimental.pallas import tpu_sc as plsc

N = 10000
E = 320000
D = 128
C = 60

NC = 2
NS = 16
NW = NC * NS
PER_W = E // NW
CHUNK = 200
NSTEPS = PER_W // CHUNK
NPAD = 10240
ZCH = 160
ROWS_PER_TILE = NPAD // NS

_SC_MESH = plsc.VectorSubcoreMesh(core_axis_name="c", subcore_axis_name="s")


@functools.partial(
    pl.kernel,
    out_type=jax.ShapeDtypeStruct((2, 2, NPAD, 16), jnp.float32),
    mesh=_SC_MESH,
    scratch_types=[
        pltpu.VMEM_SHARED((NPAD, 16), jnp.float32),
        pltpu.VMEM_SHARED((NPAD, 16), jnp.float32),
        pltpu.VMEM((CHUNK,), jnp.int32),
        pltpu.VMEM((CHUNK,), jnp.int32),
        pltpu.VMEM((CHUNK, 16), jnp.float32),
        pltpu.VMEM((ROWS_PER_TILE, 16), jnp.float32),
    ],
    compiler_params=pltpu.CompilerParams(use_tc_tiling_on_sc=False),
)
def _sc_degrees(src_hbm, dst_hbm, ones_hbm, zeros_hbm, out_hbm,
                deg_s, deg_d, sbuf, dbuf, obuf, zbuf):
    c = lax.axis_index("c")
    s = lax.axis_index("s")
    wid = c * NS + s

    pltpu.sync_copy(zeros_hbm, zbuf)
    r0 = s * ROWS_PER_TILE
    pltpu.sync_copy(zbuf, deg_s.at[pl.ds(r0, ROWS_PER_TILE)])
    pltpu.sync_copy(zbuf, deg_d.at[pl.ds(r0, ROWS_PER_TILE)])
    pltpu.sync_copy(ones_hbm, obuf)
    plsc.subcore_barrier()

    def step(j, carry):
        off = pl.multiple_of(wid * PER_W + j * CHUNK, 8)
        pltpu.sync_copy(src_hbm.at[pl.ds(off, CHUNK)], sbuf)
        pltpu.sync_copy(dst_hbm.at[pl.ds(off, CHUNK)], dbuf)
        pltpu.sync_copy(obuf, deg_s.at[sbuf], add=True)
        pltpu.sync_copy(obuf, deg_d.at[dbuf], add=True)
        return carry

    lax.fori_loop(0, NSTEPS, step, 0)
    plsc.subcore_barrier()

    pltpu.sync_copy(deg_s.at[pl.ds(r0, ROWS_PER_TILE)], zbuf)
    pltpu.sync_copy(zbuf, out_hbm.at[c, 0, pl.ds(r0, ROWS_PER_TILE)])
    pltpu.sync_copy(deg_d.at[pl.ds(r0, ROWS_PER_TILE)], zbuf)
    pltpu.sync_copy(zbuf, out_hbm.at[c, 1, pl.ds(r0, ROWS_PER_TILE)])


@functools.partial(
    pl.kernel,
    out_type=jax.ShapeDtypeStruct((2, NPAD, D), jnp.float32),
    mesh=_SC_MESH,
    scratch_types=[
        pltpu.VMEM_SHARED((NPAD, D), jnp.float32),
        pltpu.VMEM((CHUNK,), jnp.int32),
        pltpu.VMEM((CHUNK,), jnp.int32),
        pltpu.VMEM((CHUNK, D), jnp.float32),
        pltpu.SemaphoreType.DMA,
    ],
)
def _sc_aggregate(feat_hbm, src_hbm, dst_hbm, zeros_hbm, out_hbm,
                  agg, sbuf, dbuf, rows, sem):
    c = lax.axis_index("c")
    s = lax.axis_index("s")
    wid = c * NS + s

    pltpu.sync_copy(zeros_hbm, rows.at[pl.ds(0, ZCH)])
    for t in range(ROWS_PER_TILE // ZCH):
        pltpu.sync_copy(rows.at[pl.ds(0, ZCH)],
                        agg.at[pl.ds(s * ROWS_PER_TILE + t * ZCH, ZCH)])
    plsc.subcore_barrier()

    def step(j, carry):
        off = pl.multiple_of(wid * PER_W + j * CHUNK, 8)
        pltpu.sync_copy(src_hbm.at[pl.ds(off, CHUNK)], sbuf)
        pltpu.sync_copy(dst_hbm.at[pl.ds(off, CHUNK)], dbuf)
        pltpu.async_copy(feat_hbm.at[sbuf], rows, sem).wait()
        pltpu.sync_copy(rows, agg.at[dbuf], add=True)
        return carry

    lax.fori_loop(0, NSTEPS, step, 0)
    plsc.subcore_barrier()

    for t in range(ROWS_PER_TILE // ZCH):
        r0 = s * ROWS_PER_TILE + t * ZCH
        pltpu.sync_copy(agg.at[pl.ds(r0, ZCH)], rows.at[pl.ds(0, ZCH)])
        pltpu.sync_copy(rows.at[pl.ds(0, ZCH)], out_hbm.at[c, pl.ds(r0, ZCH)])


BLK = 1000
EPS = 1e-5


def _norm_from_deg(degp_blk, which):
    deg = degp_blk[0, which, :, 0] + degp_blk[1, which, :, 0]
    return lax.rsqrt(jnp.maximum(deg, 1.0))


def _embed_body(nf, w, b, degp, x_out, feat_out):
    x = jnp.dot(nf[...], w[...], preferred_element_type=jnp.float32) + b[...]
    x_out[...] = x
    ns = _norm_from_deg(degp[...], 0)
    feat_out[...] = x * ns[:, None]


def _tc_embed(nf, w_emb, b_emb, degp):
    return pl.pallas_call(
        _embed_body,
        grid=(N // BLK,),
        in_specs=[
            pl.BlockSpec((BLK, D), lambda i: (i, 0)),
            pl.BlockSpec((D, D), lambda i: (0, 0)),
            pl.BlockSpec((1, D), lambda i: (0, 0)),
            pl.BlockSpec((2, 2, BLK, 16), lambda i: (0, 0, i, 0)),
        ],
        out_specs=[
            pl.BlockSpec((BLK, D), lambda i: (i, 0)),
            pl.BlockSpec((BLK, D), lambda i: (i, 0)),
        ],
        out_shape=[
            jax.ShapeDtypeStruct((N, D), jnp.float32),
            jax.ShapeDtypeStruct((N, D), jnp.float32),
        ],
    )(nf, w_emb, b_emb, degp)


def _post_a_body(aggp, x, degp, w, b, h_out, st_out, acc):
    i = pl.program_id(0)
    nd = _norm_from_deg(degp[...], 1)
    a = (aggp[0] + aggp[1]) * nd[:, None]
    h = x[...] + jnp.dot(a, w[...], preferred_element_type=jnp.float32) + b[...]
    h_out[...] = h

    @pl.when(i == 0)
    def _():
        acc[...] = jnp.zeros_like(acc)

    acc[0, :] += jnp.sum(h, axis=0)
    acc[1, :] += jnp.sum(h * h, axis=0)

    @pl.when(i == pl.num_programs(0) - 1)
    def _():
        st_out[...] = acc[...]


def _tc_post_a(aggp, x, degp, w, b):
    return pl.pallas_call(
        _post_a_body,
        grid=(N // BLK,),
        in_specs=[
            pl.BlockSpec((2, BLK, D), lambda i: (0, i, 0)),
            pl.BlockSpec((BLK, D), lambda i: (i, 0)),
            pl.BlockSpec((2, 2, BLK, 16), lambda i: (0, 0, i, 0)),
            pl.BlockSpec((D, D), lambda i: (0, 0)),
            pl.BlockSpec((1, D), lambda i: (0, 0)),
        ],
        out_specs=[
            pl.BlockSpec((BLK, D), lambda i: (i, 0)),
            pl.BlockSpec((2, D), lambda i: (0, 0)),
        ],
        out_shape=[
            jax.ShapeDtypeStruct((N, D), jnp.float32),
            jax.ShapeDtypeStruct((2, D), jnp.float32),
        ],
        scratch_shapes=[pltpu.VMEM((2, D), jnp.float32)],
    )(aggp, x, degp, w, b)


def _bn_relu(h, st, g, be):
    mean = st[0, :] / N
    var = st[1, :] / N - mean * mean
    inv = lax.rsqrt(var + EPS)
    hn = (h - mean) * inv * g + be
    return jnp.maximum(hn, 0.0)


def _post_b_body(h1, st, g, be, w, b, degp, feat_out):
    r = _bn_relu(h1[...], st[...], g[...], be[...])
    t = jnp.dot(r, w[...], preferred_element_type=jnp.float32) + b[...]
    ns = _norm_from_deg(degp[...], 0)
    feat_out[...] = t * ns[:, None]


def _tc_post_b(h1, st, g, be, w, b, degp):
    return pl.pallas_call(
        _post_b_body,
        grid=(N // BLK,),
        in_specs=[
            pl.BlockSpec((BLK, D), lambda i: (i, 0)),
            pl.BlockSpec((2, D), lambda i: (0, 0)),
            pl.BlockSpec((1, D), lambda i: (0, 0)),
            pl.BlockSpec((1, D), lambda i: (0, 0)),
            pl.BlockSpec((D, D), lambda i: (0, 0)),
            pl.BlockSpec((1, D), lambda i: (0, 0)),
            pl.BlockSpec((2, 2, BLK, 16), lambda i: (0, 0, i, 0)),
        ],
        out_specs=pl.BlockSpec((BLK, D), lambda i: (i, 0)),
        out_shape=jax.ShapeDtypeStruct((N, D), jnp.float32),
    )(h1, st, g, be, w, b, degp)


def _final_body(h2, st, g, be, w, b, w_cls, b_cls, h_out, label_out, acc):
    i = pl.program_id(0)
    r = _bn_relu(h2[...], st[...], g[...], be[...])
    ho = jnp.dot(r, w[...], preferred_element_type=jnp.float32) + b[...]
    h_out[...] = ho

    @pl.when(i == 0)
    def _():
        acc[...] = jnp.zeros_like(acc)

    acc[0, :] += jnp.sum(ho, axis=0)

    @pl.when(i == pl.num_programs(0) - 1)
    def _():
        y = acc[...] / N
        label_out[...] = (
            jnp.dot(y, w_cls[...], preferred_element_type=jnp.float32)
            + b_cls[...]
        )


def _tc_final(h2, st, g, be, w, b, w_cls, b_cls):
    return pl.pallas_call(
        _final_body,
        grid=(N // BLK,),
        in_specs=[
            pl.BlockSpec((BLK, D), lambda i: (i, 0)),
            pl.BlockSpec((2, D), lambda i: (0, 0)),
            pl.BlockSpec((1, D), lambda i: (0, 0)),
            pl.BlockSpec((1, D), lambda i: (0, 0)),
            pl.BlockSpec((D, D), lambda i: (0, 0)),
            pl.BlockSpec((1, D), lambda i: (0, 0)),
            pl.BlockSpec((D, C), lambda i: (0, 0)),
            pl.BlockSpec((1, C), lambda i: (0, 0)),
        ],
        out_specs=[
            pl.BlockSpec((BLK, D), lambda i: (i, 0)),
            pl.BlockSpec((1, C), lambda i: (0, 0)),
        ],
        out_shape=[
            jax.ShapeDtypeStruct((N, D), jnp.float32),
            jax.ShapeDtypeStruct((1, C), jnp.float32),
        ],
        scratch_shapes=[pltpu.VMEM((1, D), jnp.float32)],
    )(h2, st, g, be, w, b, w_cls, b_cls)


def kernel(node_features, edge_index, W_emb, b_emb, W_c1, b_c1, g1, be1,
           W_l2, b_l2, W_c2, b_c2, g2, be2, W_l3, b_l3, W_cls, b_cls):
    src1d = edge_index[0]
    dst1d = edge_index[1]
    ones16 = jnp.ones((CHUNK, 16), jnp.float32)
    zeros16 = jnp.zeros((ROWS_PER_TILE, 16), jnp.float32)
    zeros128 = jnp.zeros((ZCH, D), jnp.float32)

    degp = _sc_degrees(src1d, dst1d, ones16, zeros16)
    x, feat1 = _tc_embed(node_features, W_emb, b_emb.reshape(1, D), degp)
    agg1 = _sc_aggregate(feat1, src1d, dst1d, zeros128)
    h1, st1 = _tc_post_a(agg1, x, degp, W_c1, b_c1.reshape(1, D))
    feat2 = _tc_post_b(h1, st1, g1.reshape(1, D), be1.reshape(1, D),
                       W_l2, b_l2.reshape(1, D), degp)
    agg2 = _sc_aggregate(feat2, src1d, dst1d, zeros128)
    h2, st2 = _tc_post_a(agg2, x, degp, W_c2, b_c2.reshape(1, D))
    h_out, label = _tc_final(h2, st2, g2.reshape(1, D), be2.reshape(1, D),
                             W_l3, b_l3.reshape(1, D), W_cls,
                             b_cls.reshape(1, C))
    return (h_out, label)


# speedup vs baseline: 8.8919x; 1.3066x over previous
import functools

import jax
import jax.numpy as jnp
from jax import lax
from jax.experimental import pallas as pl
from jax.experimental.pallas import tpu as pltpu
from jax.experimental.pallas import tpu_sc as plsc

N = 10000
E = 320000
D = 128
C = 60

NC = 2
NS = 16
NW = NC * NS
PER_W = E // NW
DCH = 200
DSTEPS = PER_W // DCH
CHUNK = 80
NSTEPS = PER_W // CHUNK
NPAD = 10240
ZCH = 80
ROWS_PER_TILE = NPAD // NS

_SC_MESH = plsc.VectorSubcoreMesh(core_axis_name="c", subcore_axis_name="s")


@functools.partial(
    pl.kernel,
    out_type=jax.ShapeDtypeStruct((2, 2, NPAD, 16), jnp.float32),
    mesh=_SC_MESH,
    scratch_types=[
        pltpu.VMEM_SHARED((NPAD, 16), jnp.float32),
        pltpu.VMEM_SHARED((NPAD, 16), jnp.float32),
        pltpu.VMEM((DSTEPS, DCH), jnp.int32),
        pltpu.VMEM((DSTEPS, DCH), jnp.int32),
        pltpu.VMEM((DCH, 16), jnp.float32),
        pltpu.VMEM((ROWS_PER_TILE, 16), jnp.float32),
        pltpu.SemaphoreType.DMA,
        pltpu.SemaphoreType.DMA,
        pltpu.SemaphoreType.DMA,
        pltpu.SemaphoreType.DMA,
    ],
    compiler_params=pltpu.CompilerParams(use_tc_tiling_on_sc=False),
)
def _sc_degrees(src_hbm, dst_hbm, ones_hbm, zeros_hbm, out_hbm,
                deg_s, deg_d, sbuf, dbuf, obuf, zbuf, m0, m1, m2, m3):
    c = lax.axis_index("c")
    s = lax.axis_index("s")
    wid = c * NS + s

    pltpu.sync_copy(zeros_hbm, zbuf)
    r0 = s * ROWS_PER_TILE
    pltpu.sync_copy(zbuf, deg_s.at[pl.ds(r0, ROWS_PER_TILE)])
    pltpu.sync_copy(zbuf, deg_d.at[pl.ds(r0, ROWS_PER_TILE)])
    pltpu.sync_copy(ones_hbm, obuf)
    pltpu.sync_copy(src_hbm.at[wid], sbuf)
    pltpu.sync_copy(dst_hbm.at[wid], dbuf)
    plsc.subcore_barrier()

    def step(k, carry):
        j0 = k * 2
        j1 = j0 + 1
        d0 = pltpu.async_copy(obuf, deg_s.at[sbuf.at[j0]], m0, add=True)
        d1 = pltpu.async_copy(obuf, deg_d.at[dbuf.at[j0]], m1, add=True)
        d2 = pltpu.async_copy(obuf, deg_s.at[sbuf.at[j1]], m2, add=True)
        d3 = pltpu.async_copy(obuf, deg_d.at[dbuf.at[j1]], m3, add=True)
        d0.wait()
        d1.wait()
        d2.wait()
        d3.wait()
        return carry

    lax.fori_loop(0, DSTEPS // 2, step, 0)
    plsc.subcore_barrier()

    pltpu.sync_copy(deg_s.at[pl.ds(r0, ROWS_PER_TILE)], zbuf)
    pltpu.sync_copy(zbuf, out_hbm.at[c, 0, pl.ds(r0, ROWS_PER_TILE)])
    pltpu.sync_copy(deg_d.at[pl.ds(r0, ROWS_PER_TILE)], zbuf)
    pltpu.sync_copy(zbuf, out_hbm.at[c, 1, pl.ds(r0, ROWS_PER_TILE)])


@functools.partial(
    pl.kernel,
    out_type=jax.ShapeDtypeStruct((2, NPAD, D), jnp.float32),
    mesh=_SC_MESH,
    scratch_types=[
        pltpu.VMEM_SHARED((NPAD, D), jnp.float32),
        pltpu.VMEM((NSTEPS, CHUNK), jnp.int32),
        pltpu.VMEM((NSTEPS, CHUNK), jnp.int32),
        pltpu.VMEM((CHUNK, D), jnp.float32),
        pltpu.VMEM((CHUNK, D), jnp.float32),
        pltpu.SemaphoreType.DMA,
        pltpu.SemaphoreType.DMA,
        pltpu.SemaphoreType.DMA,
        pltpu.SemaphoreType.DMA,
    ],
    compiler_params=pltpu.CompilerParams(use_tc_tiling_on_sc=False),
)
def _sc_aggregate(feat_hbm, src_hbm, dst_hbm, zeros_hbm, out_hbm,
                  agg, sbuf, dbuf, rows0, rows1, g0, g1, s0, s1):
    c = lax.axis_index("c")
    s = lax.axis_index("s")
    wid = c * NS + s

    pltpu.sync_copy(src_hbm.at[wid], sbuf)
    pltpu.sync_copy(dst_hbm.at[wid], dbuf)
    pltpu.sync_copy(zeros_hbm, rows0)
    for t in range(ROWS_PER_TILE // ZCH):
        pltpu.sync_copy(rows0, agg.at[pl.ds(s * ROWS_PER_TILE + t * ZCH, ZCH)])
    plsc.subcore_barrier()

    pltpu.async_copy(feat_hbm.at[sbuf.at[0]], rows0, g0)
    pltpu.async_copy(feat_hbm.at[sbuf.at[1]], rows1, g1)

    def step(k, carry):
        j0 = k * 2
        j1 = j0 + 1
        pltpu.make_async_copy(feat_hbm.at[sbuf.at[j0]], rows0, g0).wait()
        sc0 = pltpu.async_copy(rows0, agg.at[dbuf.at[j0]], s0, add=True)
        pltpu.make_async_copy(feat_hbm.at[sbuf.at[j1]], rows1, g1).wait()
        sc1 = pltpu.async_copy(rows1, agg.at[dbuf.at[j1]], s1, add=True)
        sc0.wait()
        pltpu.async_copy(feat_hbm.at[sbuf.at[j0 + 2]], rows0, g0)
        sc1.wait()

        @pl.when(j1 + 2 < NSTEPS)
        def _():
            pltpu.async_copy(feat_hbm.at[sbuf.at[j1 + 2]], rows1, g1)

        return carry

    last = NSTEPS - 1
    lax.fori_loop(0, NSTEPS // 2, step, 0)
    pltpu.make_async_copy(feat_hbm.at[sbuf.at[last]], rows0, g0).wait()
    pltpu.async_copy(rows0, agg.at[dbuf.at[last]], s0, add=True).wait()
    plsc.subcore_barrier()

    for t in range(ROWS_PER_TILE // ZCH):
        r0 = s * ROWS_PER_TILE + t * ZCH
        pltpu.sync_copy(agg.at[pl.ds(r0, ZCH)], rows0)
        pltpu.sync_copy(rows0, out_hbm.at[c, pl.ds(r0, ZCH)])


BLK = 1000
EPS = 1e-5


def _norm_from_deg(degp_blk, which):
    deg = degp_blk[0, which, :, 0] + degp_blk[1, which, :, 0]
    return lax.rsqrt(jnp.maximum(deg, 1.0))


def _embed_body(nf, w, b, degp, x_out, feat_out):
    x = jnp.dot(nf[...], w[...], preferred_element_type=jnp.float32) + b[...]
    x_out[...] = x
    ns = _norm_from_deg(degp[...], 0)
    feat_out[...] = x * ns[:, None]


def _tc_embed(nf, w_emb, b_emb, degp):
    return pl.pallas_call(
        _embed_body,
        grid=(N // BLK,),
        in_specs=[
            pl.BlockSpec((BLK, D), lambda i: (i, 0)),
            pl.BlockSpec((D, D), lambda i: (0, 0)),
            pl.BlockSpec((1, D), lambda i: (0, 0)),
            pl.BlockSpec((2, 2, BLK, 16), lambda i: (0, 0, i, 0)),
        ],
        out_specs=[
            pl.BlockSpec((BLK, D), lambda i: (i, 0)),
            pl.BlockSpec((BLK, D), lambda i: (i, 0)),
        ],
        out_shape=[
            jax.ShapeDtypeStruct((N, D), jnp.float32),
            jax.ShapeDtypeStruct((N, D), jnp.float32),
        ],
    )(nf, w_emb, b_emb, degp)


def _post_a_body(aggp, x, degp, w, b, h_out, st_out, acc):
    i = pl.program_id(0)
    nd = _norm_from_deg(degp[...], 1)
    a = (aggp[0] + aggp[1]) * nd[:, None]
    h = x[...] + jnp.dot(a, w[...], preferred_element_type=jnp.float32) + b[...]
    h_out[...] = h

    @pl.when(i == 0)
    def _():
        acc[...] = jnp.zeros_like(acc)

    acc[0, :] += jnp.sum(h, axis=0)
    acc[1, :] += jnp.sum(h * h, axis=0)

    @pl.when(i == pl.num_programs(0) - 1)
    def _():
        st_out[...] = acc[...]


def _tc_post_a(aggp, x, degp, w, b):
    return pl.pallas_call(
        _post_a_body,
        grid=(N // BLK,),
        in_specs=[
            pl.BlockSpec((2, BLK, D), lambda i: (0, i, 0)),
            pl.BlockSpec((BLK, D), lambda i: (i, 0)),
            pl.BlockSpec((2, 2, BLK, 16), lambda i: (0, 0, i, 0)),
            pl.BlockSpec((D, D), lambda i: (0, 0)),
            pl.BlockSpec((1, D), lambda i: (0, 0)),
        ],
        out_specs=[
            pl.BlockSpec((BLK, D), lambda i: (i, 0)),
            pl.BlockSpec((2, D), lambda i: (0, 0)),
        ],
        out_shape=[
            jax.ShapeDtypeStruct((N, D), jnp.float32),
            jax.ShapeDtypeStruct((2, D), jnp.float32),
        ],
        scratch_shapes=[pltpu.VMEM((2, D), jnp.float32)],
    )(aggp, x, degp, w, b)


def _bn_relu(h, st, g, be):
    mean = st[0, :] / N
    var = st[1, :] / N - mean * mean
    inv = lax.rsqrt(var + EPS)
    hn = (h - mean) * inv * g + be
    return jnp.maximum(hn, 0.0)


def _post_b_body(h1, st, g, be, w, b, degp, feat_out):
    r = _bn_relu(h1[...], st[...], g[...], be[...])
    t = jnp.dot(r, w[...], preferred_element_type=jnp.float32) + b[...]
    ns = _norm_from_deg(degp[...], 0)
    feat_out[...] = t * ns[:, None]


def _tc_post_b(h1, st, g, be, w, b, degp):
    return pl.pallas_call(
        _post_b_body,
        grid=(N // BLK,),
        in_specs=[
            pl.BlockSpec((BLK, D), lambda i: (i, 0)),
            pl.BlockSpec((2, D), lambda i: (0, 0)),
            pl.BlockSpec((1, D), lambda i: (0, 0)),
            pl.BlockSpec((1, D), lambda i: (0, 0)),
            pl.BlockSpec((D, D), lambda i: (0, 0)),
            pl.BlockSpec((1, D), lambda i: (0, 0)),
            pl.BlockSpec((2, 2, BLK, 16), lambda i: (0, 0, i, 0)),
        ],
        out_specs=pl.BlockSpec((BLK, D), lambda i: (i, 0)),
        out_shape=jax.ShapeDtypeStruct((N, D), jnp.float32),
    )(h1, st, g, be, w, b, degp)


def _final_body(h2, st, g, be, w, b, w_cls, b_cls, h_out, label_out, acc):
    i = pl.program_id(0)
    r = _bn_relu(h2[...], st[...], g[...], be[...])
    ho = jnp.dot(r, w[...], preferred_element_type=jnp.float32) + b[...]
    h_out[...] = ho

    @pl.when(i == 0)
    def _():
        acc[...] = jnp.zeros_like(acc)

    acc[0, :] += jnp.sum(ho, axis=0)

    @pl.when(i == pl.num_programs(0) - 1)
    def _():
        y = acc[...] / N
        label_out[...] = (
            jnp.dot(y, w_cls[...], preferred_element_type=jnp.float32)
            + b_cls[...]
        )


def _tc_final(h2, st, g, be, w, b, w_cls, b_cls):
    return pl.pallas_call(
        _final_body,
        grid=(N // BLK,),
        in_specs=[
            pl.BlockSpec((BLK, D), lambda i: (i, 0)),
            pl.BlockSpec((2, D), lambda i: (0, 0)),
            pl.BlockSpec((1, D), lambda i: (0, 0)),
            pl.BlockSpec((1, D), lambda i: (0, 0)),
            pl.BlockSpec((D, D), lambda i: (0, 0)),
            pl.BlockSpec((1, D), lambda i: (0, 0)),
            pl.BlockSpec((D, C), lambda i: (0, 0)),
            pl.BlockSpec((1, C), lambda i: (0, 0)),
        ],
        out_specs=[
            pl.BlockSpec((BLK, D), lambda i: (i, 0)),
            pl.BlockSpec((1, C), lambda i: (0, 0)),
        ],
        out_shape=[
            jax.ShapeDtypeStruct((N, D), jnp.float32),
            jax.ShapeDtypeStruct((1, C), jnp.float32),
        ],
        scratch_shapes=[pltpu.VMEM((1, D), jnp.float32)],
    )(h2, st, g, be, w, b, w_cls, b_cls)


def kernel(node_features, edge_index, W_emb, b_emb, W_c1, b_c1, g1, be1,
           W_l2, b_l2, W_c2, b_c2, g2, be2, W_l3, b_l3, W_cls, b_cls):
    src_d = edge_index[0].reshape(NW, DSTEPS, DCH)
    dst_d = edge_index[1].reshape(NW, DSTEPS, DCH)
    src_c = edge_index[0].reshape(NW, NSTEPS, CHUNK)
    dst_c = edge_index[1].reshape(NW, NSTEPS, CHUNK)
    ones16 = jnp.ones((DCH, 16), jnp.float32)
    zeros16 = jnp.zeros((ROWS_PER_TILE, 16), jnp.float32)
    zeros128 = jnp.zeros((ZCH, D), jnp.float32)

    degp = _sc_degrees(src_d, dst_d, ones16, zeros16)
    x, feat1 = _tc_embed(node_features, W_emb, b_emb.reshape(1, D), degp)
    agg1 = _sc_aggregate(feat1, src_c, dst_c, zeros128)
    h1, st1 = _tc_post_a(agg1, x, degp, W_c1, b_c1.reshape(1, D))
    feat2 = _tc_post_b(h1, st1, g1.reshape(1, D), be1.reshape(1, D),
                       W_l2, b_l2.reshape(1, D), degp)
    agg2 = _sc_aggregate(feat2, src_c, dst_c, zeros128)
    h2, st2 = _tc_post_a(agg2, x, degp, W_c2, b_c2.reshape(1, D))
    h_out, label = _tc_final(h2, st2, g2.reshape(1, D), be2.reshape(1, D),
                             W_l3, b_l3.reshape(1, D), W_cls,
                             b_cls.reshape(1, C))
    return (h_out, label)


# speedup vs baseline: 10.4250x; 1.1724x over previous
import functools

import jax
import jax.numpy as jnp
from jax import lax
from jax.experimental import pallas as pl
from jax.experimental.pallas import tpu as pltpu
from jax.experimental.pallas import tpu_sc as plsc

N = 10000
E = 320000
D = 128
C = 60

NC = 2
NS = 16
NW = NC * NS
PER_W = E // NW
DCH = 200
DSTEPS = PER_W // DCH
CHUNK = 40
NSTEPS = PER_W // CHUNK
NBUF = 4
NPAD = 10240
ZCH = 40
ROWS_PER_TILE = NPAD // NS

_SC_MESH = plsc.VectorSubcoreMesh(core_axis_name="c", subcore_axis_name="s")


@functools.partial(
    pl.kernel,
    out_type=jax.ShapeDtypeStruct((2, 2, NPAD, 16), jnp.float32),
    mesh=_SC_MESH,
    scratch_types=[
        pltpu.VMEM_SHARED((NPAD, 16), jnp.float32),
        pltpu.VMEM_SHARED((NPAD, 16), jnp.float32),
        pltpu.VMEM((DSTEPS, DCH), jnp.int32),
        pltpu.VMEM((DSTEPS, DCH), jnp.int32),
        pltpu.VMEM((DCH, 16), jnp.float32),
        pltpu.VMEM((ROWS_PER_TILE, 16), jnp.float32),
        pltpu.SemaphoreType.DMA,
        pltpu.SemaphoreType.DMA,
        pltpu.SemaphoreType.DMA,
        pltpu.SemaphoreType.DMA,
    ],
    compiler_params=pltpu.CompilerParams(use_tc_tiling_on_sc=False),
)
def _sc_degrees(src_hbm, dst_hbm, ones_hbm, zeros_hbm, out_hbm,
                deg_s, deg_d, sbuf, dbuf, obuf, zbuf, m0, m1, m2, m3):
    c = lax.axis_index("c")
    s = lax.axis_index("s")
    wid = c * NS + s

    pltpu.sync_copy(zeros_hbm, zbuf)
    r0 = s * ROWS_PER_TILE
    pltpu.sync_copy(zbuf, deg_s.at[pl.ds(r0, ROWS_PER_TILE)])
    pltpu.sync_copy(zbuf, deg_d.at[pl.ds(r0, ROWS_PER_TILE)])
    pltpu.sync_copy(ones_hbm, obuf)
    pltpu.sync_copy(src_hbm.at[wid], sbuf)
    pltpu.sync_copy(dst_hbm.at[wid], dbuf)
    plsc.subcore_barrier()

    def step(k, carry):
        j0 = k * 2
        j1 = j0 + 1
        d0 = pltpu.async_copy(obuf, deg_s.at[sbuf.at[j0]], m0, add=True)
        d1 = pltpu.async_copy(obuf, deg_d.at[dbuf.at[j0]], m1, add=True)
        d2 = pltpu.async_copy(obuf, deg_s.at[sbuf.at[j1]], m2, add=True)
        d3 = pltpu.async_copy(obuf, deg_d.at[dbuf.at[j1]], m3, add=True)
        d0.wait()
        d1.wait()
        d2.wait()
        d3.wait()
        return carry

    lax.fori_loop(0, DSTEPS // 2, step, 0)
    plsc.subcore_barrier()

    pltpu.sync_copy(deg_s.at[pl.ds(r0, ROWS_PER_TILE)], zbuf)
    pltpu.sync_copy(zbuf, out_hbm.at[c, 0, pl.ds(r0, ROWS_PER_TILE)])
    pltpu.sync_copy(deg_d.at[pl.ds(r0, ROWS_PER_TILE)], zbuf)
    pltpu.sync_copy(zbuf, out_hbm.at[c, 1, pl.ds(r0, ROWS_PER_TILE)])


@functools.partial(
    pl.kernel,
    out_type=jax.ShapeDtypeStruct((2, NPAD, D), jnp.float32),
    mesh=_SC_MESH,
    scratch_types=[
        pltpu.VMEM_SHARED((NPAD, D), jnp.float32),
        pltpu.VMEM((NSTEPS, CHUNK), jnp.int32),
        pltpu.VMEM((NSTEPS, CHUNK), jnp.int32),
        [pltpu.VMEM((CHUNK, D), jnp.float32) for _ in range(NBUF)],
        [pltpu.SemaphoreType.DMA for _ in range(NBUF)],
        [pltpu.SemaphoreType.DMA for _ in range(NBUF)],
    ],
    compiler_params=pltpu.CompilerParams(use_tc_tiling_on_sc=False),
)
def _sc_aggregate(feat_hbm, src_hbm, dst_hbm, zeros_hbm, out_hbm,
                  agg, sbuf, dbuf, rows, gs, ss):
    c = lax.axis_index("c")
    s = lax.axis_index("s")
    wid = c * NS + s

    pltpu.sync_copy(src_hbm.at[wid], sbuf)
    pltpu.sync_copy(dst_hbm.at[wid], dbuf)
    pltpu.sync_copy(zeros_hbm, rows[0])
    for t in range(ROWS_PER_TILE // ZCH):
        pltpu.sync_copy(rows[0],
                        agg.at[pl.ds(s * ROWS_PER_TILE + t * ZCH, ZCH)])
    plsc.subcore_barrier()

    for b in range(NBUF):
        pltpu.async_copy(feat_hbm.at[sbuf.at[b]], rows[b], gs[b])

    def step(k, carry):
        j = k * NBUF
        for b in range(NBUF):
            jb = j + b
            pltpu.make_async_copy(feat_hbm.at[sbuf.at[jb]], rows[b],
                                  gs[b]).wait()
            pltpu.async_copy(rows[b], agg.at[dbuf.at[jb]], ss[b], add=True)
        for b in range(NBUF):
            jb = j + b
            pltpu.make_async_copy(rows[b], agg.at[dbuf.at[jb]], ss[b]).wait()
            nxt = jb + NBUF

            @pl.when(nxt < NSTEPS)
            def _():
                pltpu.async_copy(feat_hbm.at[sbuf.at[nxt]], rows[b], gs[b])

        return carry

    lax.fori_loop(0, NSTEPS // NBUF, step, 0)
    for b in range(NSTEPS % NBUF):
        jb = (NSTEPS // NBUF) * NBUF + b
        pltpu.make_async_copy(feat_hbm.at[sbuf.at[jb]], rows[b], gs[b]).wait()
        pltpu.async_copy(rows[b], agg.at[dbuf.at[jb]], ss[b], add=True).wait()
    plsc.subcore_barrier()

    for t in range(ROWS_PER_TILE // ZCH):
        r0 = s * ROWS_PER_TILE + t * ZCH
        pltpu.sync_copy(agg.at[pl.ds(r0, ZCH)], rows[0])
        pltpu.sync_copy(rows[0], out_hbm.at[c, pl.ds(r0, ZCH)])


BLK = 1000
EPS = 1e-5


def _norm_from_deg(degp_blk, which):
    deg = degp_blk[0, which, :, 0] + degp_blk[1, which, :, 0]
    return lax.rsqrt(jnp.maximum(deg, 1.0))


def _embed_body(nf, w, b, degp, x_out, feat_out):
    x = jnp.dot(nf[...], w[...], preferred_element_type=jnp.float32) + b[...]
    x_out[...] = x
    ns = _norm_from_deg(degp[...], 0)
    feat_out[...] = x * ns[:, None]


def _tc_embed(nf, w_emb, b_emb, degp):
    return pl.pallas_call(
        _embed_body,
        grid=(N // BLK,),
        in_specs=[
            pl.BlockSpec((BLK, D), lambda i: (i, 0)),
            pl.BlockSpec((D, D), lambda i: (0, 0)),
            pl.BlockSpec((1, D), lambda i: (0, 0)),
            pl.BlockSpec((2, 2, BLK, 16), lambda i: (0, 0, i, 0)),
        ],
        out_specs=[
            pl.BlockSpec((BLK, D), lambda i: (i, 0)),
            pl.BlockSpec((BLK, D), lambda i: (i, 0)),
        ],
        out_shape=[
            jax.ShapeDtypeStruct((N, D), jnp.float32),
            jax.ShapeDtypeStruct((N, D), jnp.float32),
        ],
    )(nf, w_emb, b_emb, degp)


def _post_a_body(aggp, x, degp, w, b, h_out, st_out, acc):
    i = pl.program_id(0)
    nd = _norm_from_deg(degp[...], 1)
    a = (aggp[0] + aggp[1]) * nd[:, None]
    h = x[...] + jnp.dot(a, w[...], preferred_element_type=jnp.float32) + b[...]
    h_out[...] = h

    @pl.when(i == 0)
    def _():
        acc[...] = jnp.zeros_like(acc)

    acc[0, :] += jnp.sum(h, axis=0)
    acc[1, :] += jnp.sum(h * h, axis=0)

    @pl.when(i == pl.num_programs(0) - 1)
    def _():
        st_out[...] = acc[...]


def _tc_post_a(aggp, x, degp, w, b):
    return pl.pallas_call(
        _post_a_body,
        grid=(N // BLK,),
        in_specs=[
            pl.BlockSpec((2, BLK, D), lambda i: (0, i, 0)),
            pl.BlockSpec((BLK, D), lambda i: (i, 0)),
            pl.BlockSpec((2, 2, BLK, 16), lambda i: (0, 0, i, 0)),
            pl.BlockSpec((D, D), lambda i: (0, 0)),
            pl.BlockSpec((1, D), lambda i: (0, 0)),
        ],
        out_specs=[
            pl.BlockSpec((BLK, D), lambda i: (i, 0)),
            pl.BlockSpec((2, D), lambda i: (0, 0)),
        ],
        out_shape=[
            jax.ShapeDtypeStruct((N, D), jnp.float32),
            jax.ShapeDtypeStruct((2, D), jnp.float32),
        ],
        scratch_shapes=[pltpu.VMEM((2, D), jnp.float32)],
    )(aggp, x, degp, w, b)


def _bn_relu(h, st, g, be):
    mean = st[0, :] / N
    var = st[1, :] / N - mean * mean
    inv = lax.rsqrt(var + EPS)
    hn = (h - mean) * inv * g + be
    return jnp.maximum(hn, 0.0)


def _post_b_body(h1, st, g, be, w, b, degp, feat_out):
    r = _bn_relu(h1[...], st[...], g[...], be[...])
    t = jnp.dot(r, w[...], preferred_element_type=jnp.float32) + b[...]
    ns = _norm_from_deg(degp[...], 0)
    feat_out[...] = t * ns[:, None]


def _tc_post_b(h1, st, g, be, w, b, degp):
    return pl.pallas_call(
        _post_b_body,
        grid=(N // BLK,),
        in_specs=[
            pl.BlockSpec((BLK, D), lambda i: (i, 0)),
            pl.BlockSpec((2, D), lambda i: (0, 0)),
            pl.BlockSpec((1, D), lambda i: (0, 0)),
            pl.BlockSpec((1, D), lambda i: (0, 0)),
            pl.BlockSpec((D, D), lambda i: (0, 0)),
            pl.BlockSpec((1, D), lambda i: (0, 0)),
            pl.BlockSpec((2, 2, BLK, 16), lambda i: (0, 0, i, 0)),
        ],
        out_specs=pl.BlockSpec((BLK, D), lambda i: (i, 0)),
        out_shape=jax.ShapeDtypeStruct((N, D), jnp.float32),
    )(h1, st, g, be, w, b, degp)


def _final_body(h2, st, g, be, w, b, w_cls, b_cls, h_out, label_out, acc):
    i = pl.program_id(0)
    r = _bn_relu(h2[...], st[...], g[...], be[...])
    ho = jnp.dot(r, w[...], preferred_element_type=jnp.float32) + b[...]
    h_out[...] = ho

    @pl.when(i == 0)
    def _():
        acc[...] = jnp.zeros_like(acc)

    acc[0, :] += jnp.sum(ho, axis=0)

    @pl.when(i == pl.num_programs(0) - 1)
    def _():
        y = acc[...] / N
        label_out[...] = (
            jnp.dot(y, w_cls[...], preferred_element_type=jnp.float32)
            + b_cls[...]
        )


def _tc_final(h2, st, g, be, w, b, w_cls, b_cls):
    return pl.pallas_call(
        _final_body,
        grid=(N // BLK,),
        in_specs=[
            pl.BlockSpec((BLK, D), lambda i: (i, 0)),
            pl.BlockSpec((2, D), lambda i: (0, 0)),
            pl.BlockSpec((1, D), lambda i: (0, 0)),
            pl.BlockSpec((1, D), lambda i: (0, 0)),
            pl.BlockSpec((D, D), lambda i: (0, 0)),
            pl.BlockSpec((1, D), lambda i: (0, 0)),
            pl.BlockSpec((D, C), lambda i: (0, 0)),
            pl.BlockSpec((1, C), lambda i: (0, 0)),
        ],
        out_specs=[
            pl.BlockSpec((BLK, D), lambda i: (i, 0)),
            pl.BlockSpec((1, C), lambda i: (0, 0)),
        ],
        out_shape=[
            jax.ShapeDtypeStruct((N, D), jnp.float32),
            jax.ShapeDtypeStruct((1, C), jnp.float32),
        ],
        scratch_shapes=[pltpu.VMEM((1, D), jnp.float32)],
    )(h2, st, g, be, w, b, w_cls, b_cls)


def kernel(node_features, edge_index, W_emb, b_emb, W_c1, b_c1, g1, be1,
           W_l2, b_l2, W_c2, b_c2, g2, be2, W_l3, b_l3, W_cls, b_cls):
    src_d = edge_index[0].reshape(NW, DSTEPS, DCH)
    dst_d = edge_index[1].reshape(NW, DSTEPS, DCH)
    src_c = edge_index[0].reshape(NW, NSTEPS, CHUNK)
    dst_c = edge_index[1].reshape(NW, NSTEPS, CHUNK)
    ones16 = jnp.ones((DCH, 16), jnp.float32)
    zeros16 = jnp.zeros((ROWS_PER_TILE, 16), jnp.float32)
    zeros128 = jnp.zeros((ZCH, D), jnp.float32)

    degp = _sc_degrees(src_d, dst_d, ones16, zeros16)
    x, feat1 = _tc_embed(node_features, W_emb, b_emb.reshape(1, D), degp)
    agg1 = _sc_aggregate(feat1, src_c, dst_c, zeros128)
    h1, st1 = _tc_post_a(agg1, x, degp, W_c1, b_c1.reshape(1, D))
    feat2 = _tc_post_b(h1, st1, g1.reshape(1, D), be1.reshape(1, D),
                       W_l2, b_l2.reshape(1, D), degp)
    agg2 = _sc_aggregate(feat2, src_c, dst_c, zeros128)
    h2, st2 = _tc_post_a(agg2, x, degp, W_c2, b_c2.reshape(1, D))
    h_out, label = _tc_final(h2, st2, g2.reshape(1, D), be2.reshape(1, D),
                             W_l3, b_l3.reshape(1, D), W_cls,
                             b_cls.reshape(1, C))
    return (h_out, label)


# speedup vs baseline: 10.6030x; 1.0171x over previous
import functools

import jax
import jax.numpy as jnp
from jax import lax
from jax.experimental import pallas as pl
from jax.experimental.pallas import tpu as pltpu
from jax.experimental.pallas import tpu_sc as plsc

N = 10000
E = 320000
D = 128
C = 60

NC = 2
NS = 16
NW = NC * NS
PER_W = E // NW
DCH = 200
DSTEPS = PER_W // DCH
CHUNK = 40
NSTEPS = PER_W // CHUNK
NBUF = 4
NPAD = 10240
ZCH = 40
ROWS_PER_TILE = NPAD // NS

_SC_MESH = plsc.VectorSubcoreMesh(core_axis_name="c", subcore_axis_name="s")


@functools.partial(
    pl.kernel,
    out_type=jax.ShapeDtypeStruct((2, 2, NPAD, 16), jnp.float32),
    mesh=_SC_MESH,
    scratch_types=[
        pltpu.VMEM_SHARED((NPAD, 16), jnp.float32),
        pltpu.VMEM_SHARED((NPAD, 16), jnp.float32),
        pltpu.VMEM((DSTEPS, DCH), jnp.int32),
        pltpu.VMEM((DSTEPS, DCH), jnp.int32),
        pltpu.VMEM((DCH, 16), jnp.float32),
        pltpu.VMEM((ROWS_PER_TILE, 16), jnp.float32),
        pltpu.SemaphoreType.DMA,
        pltpu.SemaphoreType.DMA,
        pltpu.SemaphoreType.DMA,
        pltpu.SemaphoreType.DMA,
    ],
    compiler_params=pltpu.CompilerParams(use_tc_tiling_on_sc=False),
)
def _sc_degrees(src_hbm, dst_hbm, ones_hbm, zeros_hbm, out_hbm,
                deg_s, deg_d, sbuf, dbuf, obuf, zbuf, m0, m1, m2, m3):
    c = lax.axis_index("c")
    s = lax.axis_index("s")
    wid = c * NS + s

    pltpu.sync_copy(zeros_hbm, zbuf)
    r0 = s * ROWS_PER_TILE
    pltpu.sync_copy(zbuf, deg_s.at[pl.ds(r0, ROWS_PER_TILE)])
    pltpu.sync_copy(zbuf, deg_d.at[pl.ds(r0, ROWS_PER_TILE)])
    pltpu.sync_copy(ones_hbm, obuf)
    pltpu.sync_copy(src_hbm.at[wid], sbuf)
    pltpu.sync_copy(dst_hbm.at[wid], dbuf)
    plsc.subcore_barrier()

    def step(k, carry):
        j0 = k * 2
        j1 = j0 + 1
        d0 = pltpu.async_copy(obuf, deg_s.at[sbuf.at[j0]], m0, add=True)
        d1 = pltpu.async_copy(obuf, deg_d.at[dbuf.at[j0]], m1, add=True)
        d2 = pltpu.async_copy(obuf, deg_s.at[sbuf.at[j1]], m2, add=True)
        d3 = pltpu.async_copy(obuf, deg_d.at[dbuf.at[j1]], m3, add=True)
        d0.wait()
        d1.wait()
        d2.wait()
        d3.wait()
        return carry

    lax.fori_loop(0, DSTEPS // 2, step, 0)
    plsc.subcore_barrier()

    pltpu.sync_copy(deg_s.at[pl.ds(r0, ROWS_PER_TILE)], zbuf)
    pltpu.sync_copy(zbuf, out_hbm.at[c, 0, pl.ds(r0, ROWS_PER_TILE)])
    pltpu.sync_copy(deg_d.at[pl.ds(r0, ROWS_PER_TILE)], zbuf)
    pltpu.sync_copy(zbuf, out_hbm.at[c, 1, pl.ds(r0, ROWS_PER_TILE)])


@functools.partial(
    pl.kernel,
    out_type=jax.ShapeDtypeStruct((2, NPAD, D), jnp.float32),
    mesh=_SC_MESH,
    scratch_types=[
        pltpu.VMEM_SHARED((NPAD, D), jnp.float32),
        pltpu.VMEM((NSTEPS, CHUNK), jnp.int32),
        pltpu.VMEM((NSTEPS, CHUNK), jnp.int32),
        [pltpu.VMEM((CHUNK, D), jnp.float32) for _ in range(NBUF)],
        [pltpu.SemaphoreType.DMA for _ in range(NBUF)],
        [pltpu.SemaphoreType.DMA for _ in range(NBUF)],
    ],
    compiler_params=pltpu.CompilerParams(use_tc_tiling_on_sc=False),
)
def _sc_aggregate(feat_hbm, src_hbm, dst_hbm, zeros_hbm, out_hbm,
                  agg, sbuf, dbuf, rows, gs, ss):
    c = lax.axis_index("c")
    s = lax.axis_index("s")
    wid = c * NS + s

    pltpu.sync_copy(src_hbm.at[wid], sbuf)
    pltpu.sync_copy(dst_hbm.at[wid], dbuf)
    pltpu.sync_copy(zeros_hbm, rows[0])
    for t in range(ROWS_PER_TILE // ZCH):
        pltpu.sync_copy(rows[0],
                        agg.at[pl.ds(s * ROWS_PER_TILE + t * ZCH, ZCH)])
    plsc.subcore_barrier()

    for b in range(NBUF):
        pltpu.async_copy(feat_hbm.at[sbuf.at[b]], rows[b], gs[b])

    def step(k, carry):
        j = k * NBUF
        for b in range(NBUF):
            jb = j + b
            pltpu.make_async_copy(feat_hbm.at[sbuf.at[jb]], rows[b],
                                  gs[b]).wait()
            pltpu.async_copy(rows[b], agg.at[dbuf.at[jb]], ss[b], add=True)
        for b in range(NBUF):
            jb = j + b
            pltpu.make_async_copy(rows[b], agg.at[dbuf.at[jb]], ss[b]).wait()
            nxt = jb + NBUF

            @pl.when(nxt < NSTEPS)
            def _():
                pltpu.async_copy(feat_hbm.at[sbuf.at[nxt]], rows[b], gs[b])

        return carry

    lax.fori_loop(0, NSTEPS // NBUF, step, 0)
    for b in range(NSTEPS % NBUF):
        jb = (NSTEPS // NBUF) * NBUF + b
        pltpu.make_async_copy(feat_hbm.at[sbuf.at[jb]], rows[b], gs[b]).wait()
        pltpu.async_copy(rows[b], agg.at[dbuf.at[jb]], ss[b], add=True).wait()
    plsc.subcore_barrier()

    for t in range(ROWS_PER_TILE // ZCH):
        r0 = s * ROWS_PER_TILE + t * ZCH
        pltpu.sync_copy(agg.at[pl.ds(r0, ZCH)], rows[0])
        pltpu.sync_copy(rows[0], out_hbm.at[c, pl.ds(r0, ZCH)])


BLK = 1000
EPS = 1e-5


def _norm_from_deg(degp_blk, which):
    deg = degp_blk[0, which, :, 0] + degp_blk[1, which, :, 0]
    return lax.rsqrt(jnp.maximum(deg, 1.0))


def _embed_body(nf, w, b, degp, x_out, feat_out):
    x = jnp.dot(nf[...], w[...], preferred_element_type=jnp.float32) + b[...]
    x_out[...] = x
    ns = _norm_from_deg(degp[...], 0)
    feat_out[...] = x * ns[:, None]


def _tc_embed(nf, w_emb, b_emb, degp):
    return pl.pallas_call(
        _embed_body,
        grid=(N // BLK,),
        in_specs=[
            pl.BlockSpec((BLK, D), lambda i: (i, 0)),
            pl.BlockSpec((D, D), lambda i: (0, 0)),
            pl.BlockSpec((1, D), lambda i: (0, 0)),
            pl.BlockSpec((2, 2, BLK, 16), lambda i: (0, 0, i, 0)),
        ],
        out_specs=[
            pl.BlockSpec((BLK, D), lambda i: (i, 0)),
            pl.BlockSpec((BLK, D), lambda i: (i, 0)),
        ],
        out_shape=[
            jax.ShapeDtypeStruct((N, D), jnp.float32),
            jax.ShapeDtypeStruct((N, D), jnp.float32),
        ],
    )(nf, w_emb, b_emb, degp)


NB = N // BLK


def _bn_relu_from_acc(h, acc, g, be):
    mean = acc[0, :] / N
    var = acc[1, :] / N - mean * mean
    inv = lax.rsqrt(var + EPS)
    hn = (h - mean) * inv * g + be
    return jnp.maximum(hn, 0.0)


def _mid_body(aggp, x, degp, w1, b1, g, be, w2, b2, feat_out, acc, hbuf):
    i = pl.program_id(0)

    @pl.when(i == 0)
    def _():
        acc[...] = jnp.zeros_like(acc)

    @pl.when(i < NB)
    def _():
        nd = _norm_from_deg(degp[...], 1)
        a = (aggp[0] + aggp[1]) * nd[:, None]
        h = (x[...] + jnp.dot(a, w1[...], preferred_element_type=jnp.float32)
             + b1[...])
        hbuf[pl.ds(i * BLK, BLK), :] = h
        acc[0, :] += jnp.sum(h, axis=0)
        acc[1, :] += jnp.sum(h * h, axis=0)

    @pl.when(i >= NB)
    def _():
        h = hbuf[pl.ds((i - NB) * BLK, BLK), :]
        r = _bn_relu_from_acc(h, acc[...], g[...], be[...])
        tt = jnp.dot(r, w2[...], preferred_element_type=jnp.float32) + b2[...]
        ns = _norm_from_deg(degp[...], 0)
        feat_out[...] = tt * ns[:, None]


def _tc_mid(aggp, x, degp, w1, b1, g, be, w2, b2):
    cst = lambda i: (0, 0)
    p1 = lambda i: (jnp.where(i < NB, i, NB - 1), 0)
    p1agg = lambda i: (0, jnp.where(i < NB, i, NB - 1), 0)
    both = lambda i: (0, 0, i % NB, 0)
    return pl.pallas_call(
        _mid_body,
        grid=(2 * NB,),
        in_specs=[
            pl.BlockSpec((2, BLK, D), p1agg),
            pl.BlockSpec((BLK, D), p1),
            pl.BlockSpec((2, 2, BLK, 16), both),
            pl.BlockSpec((D, D), cst),
            pl.BlockSpec((1, D), cst),
            pl.BlockSpec((1, D), cst),
            pl.BlockSpec((1, D), cst),
            pl.BlockSpec((D, D), cst),
            pl.BlockSpec((1, D), cst),
        ],
        out_specs=pl.BlockSpec((BLK, D), lambda i: (jnp.where(i < NB, 0,
                                                              i - NB), 0)),
        out_shape=jax.ShapeDtypeStruct((N, D), jnp.float32),
        scratch_shapes=[pltpu.VMEM((2, D), jnp.float32),
                        pltpu.VMEM((N, D), jnp.float32)],
    )(aggp, x, degp, w1, b1, g, be, w2, b2)


def _fin_body(aggp, x, degp, w1, b1, g, be, w3, b3, wc, bc,
              hf_out, label_out, acc, hbuf):
    i = pl.program_id(0)

    @pl.when(i == 0)
    def _():
        acc[...] = jnp.zeros_like(acc)

    @pl.when(i < NB)
    def _():
        nd = _norm_from_deg(degp[...], 1)
        a = (aggp[0] + aggp[1]) * nd[:, None]
        h = (x[...] + jnp.dot(a, w1[...], preferred_element_type=jnp.float32)
             + b1[...])
        hbuf[pl.ds(i * BLK, BLK), :] = h
        acc[0, :] += jnp.sum(h, axis=0)
        acc[1, :] += jnp.sum(h * h, axis=0)

    @pl.when(i >= NB)
    def _():
        h = hbuf[pl.ds((i - NB) * BLK, BLK), :]
        r = _bn_relu_from_acc(h, acc[...], g[...], be[...])
        ho = jnp.dot(r, w3[...], preferred_element_type=jnp.float32) + b3[...]
        hf_out[...] = ho
        acc[2, :] += jnp.sum(ho, axis=0)

    @pl.when(i == 2 * NB - 1)
    def _():
        y = acc[2:3, :] / N
        label_out[...] = (
            jnp.dot(y, wc[...], preferred_element_type=jnp.float32) + bc[...]
        )


def _tc_fin(aggp, x, degp, w1, b1, g, be, w3, b3, wc, bc):
    cst = lambda i: (0, 0)
    p1 = lambda i: (jnp.where(i < NB, i, NB - 1), 0)
    p1agg = lambda i: (0, jnp.where(i < NB, i, NB - 1), 0)
    both = lambda i: (0, 0, i % NB, 0)
    return pl.pallas_call(
        _fin_body,
        grid=(2 * NB,),
        in_specs=[
            pl.BlockSpec((2, BLK, D), p1agg),
            pl.BlockSpec((BLK, D), p1),
            pl.BlockSpec((2, 2, BLK, 16), both),
            pl.BlockSpec((D, D), cst),
            pl.BlockSpec((1, D), cst),
            pl.BlockSpec((1, D), cst),
            pl.BlockSpec((1, D), cst),
            pl.BlockSpec((D, D), cst),
            pl.BlockSpec((1, D), cst),
            pl.BlockSpec((D, C), cst),
            pl.BlockSpec((1, C), cst),
        ],
        out_specs=[
            pl.BlockSpec((BLK, D), lambda i: (jnp.where(i < NB, 0, i - NB),
                                              0)),
            pl.BlockSpec((1, C), cst),
        ],
        out_shape=[
            jax.ShapeDtypeStruct((N, D), jnp.float32),
            jax.ShapeDtypeStruct((1, C), jnp.float32),
        ],
        scratch_shapes=[pltpu.VMEM((3, D), jnp.float32),
                        pltpu.VMEM((N, D), jnp.float32)],
    )(aggp, x, degp, w1, b1, g, be, w3, b3, wc, bc)


def kernel(node_features, edge_index, W_emb, b_emb, W_c1, b_c1, g1, be1,
           W_l2, b_l2, W_c2, b_c2, g2, be2, W_l3, b_l3, W_cls, b_cls):
    src_d = edge_index[0].reshape(NW, DSTEPS, DCH)
    dst_d = edge_index[1].reshape(NW, DSTEPS, DCH)
    src_c = edge_index[0].reshape(NW, NSTEPS, CHUNK)
    dst_c = edge_index[1].reshape(NW, NSTEPS, CHUNK)
    ones16 = jnp.ones((DCH, 16), jnp.float32)
    zeros16 = jnp.zeros((ROWS_PER_TILE, 16), jnp.float32)
    zeros128 = jnp.zeros((ZCH, D), jnp.float32)

    degp = _sc_degrees(src_d, dst_d, ones16, zeros16)
    x, feat1 = _tc_embed(node_features, W_emb, b_emb.reshape(1, D), degp)
    agg1 = _sc_aggregate(feat1, src_c, dst_c, zeros128)
    feat2 = _tc_mid(agg1, x, degp, W_c1, b_c1.reshape(1, D),
                    g1.reshape(1, D), be1.reshape(1, D),
                    W_l2, b_l2.reshape(1, D))
    agg2 = _sc_aggregate(feat2, src_c, dst_c, zeros128)
    h_out, label = _tc_fin(agg2, x, degp, W_c2, b_c2.reshape(1, D),
                           g2.reshape(1, D), be2.reshape(1, D),
                           W_l3, b_l3.reshape(1, D), W_cls,
                           b_cls.reshape(1, C))
    return (h_out, label)


# speedup vs baseline: 10.8748x; 1.0256x over previous
import functools

import jax
import jax.numpy as jnp
from jax import lax
from jax.experimental import pallas as pl
from jax.experimental.pallas import tpu as pltpu
from jax.experimental.pallas import tpu_sc as plsc

N = 10000
E = 320000
D = 128
C = 60

NC = 2
NS = 16
NW = NC * NS
PER_W = E // NW
DCH = 200
DSTEPS = PER_W // DCH
CHUNK = 40
NSTEPS = PER_W // CHUNK
NBUF = 5
NPAD = 10240
ZCH = 40
ROWS_PER_TILE = NPAD // NS

_SC_MESH = plsc.VectorSubcoreMesh(core_axis_name="c", subcore_axis_name="s")


@functools.partial(
    pl.kernel,
    out_type=jax.ShapeDtypeStruct((2, 2, NPAD, 16), jnp.float32),
    mesh=_SC_MESH,
    scratch_types=[
        pltpu.VMEM_SHARED((NPAD, 16), jnp.float32),
        pltpu.VMEM_SHARED((NPAD, 16), jnp.float32),
        pltpu.VMEM((PER_W,), jnp.int32),
        pltpu.VMEM((PER_W,), jnp.int32),
        pltpu.VMEM((DCH, 16), jnp.float32),
        pltpu.VMEM((ROWS_PER_TILE, 16), jnp.float32),
        pltpu.SemaphoreType.DMA,
        pltpu.SemaphoreType.DMA,
        pltpu.SemaphoreType.DMA,
        pltpu.SemaphoreType.DMA,
    ],
    compiler_params=pltpu.CompilerParams(use_tc_tiling_on_sc=False),
)
def _sc_degrees(src_hbm, dst_hbm, ones_hbm, zeros_hbm, out_hbm,
                deg_s, deg_d, sbuf, dbuf, obuf, zbuf, m0, m1, m2, m3):
    c = lax.axis_index("c")
    s = lax.axis_index("s")
    wid = c * NS + s

    pltpu.sync_copy(zeros_hbm, zbuf)
    r0 = s * ROWS_PER_TILE
    pltpu.sync_copy(zbuf, deg_s.at[pl.ds(r0, ROWS_PER_TILE)])
    pltpu.sync_copy(zbuf, deg_d.at[pl.ds(r0, ROWS_PER_TILE)])
    pltpu.sync_copy(ones_hbm, obuf)
    base = wid * PER_W
    pltpu.sync_copy(src_hbm.at[pl.ds(base, PER_W)], sbuf)
    pltpu.sync_copy(dst_hbm.at[pl.ds(base, PER_W)], dbuf)
    plsc.subcore_barrier()

    def step(k, carry):
        o0 = k * 2 * DCH
        o1 = o0 + DCH
        d0 = pltpu.async_copy(obuf, deg_s.at[sbuf.at[pl.ds(o0, DCH)]], m0,
                              add=True)
        d1 = pltpu.async_copy(obuf, deg_d.at[dbuf.at[pl.ds(o0, DCH)]], m1,
                              add=True)
        d2 = pltpu.async_copy(obuf, deg_s.at[sbuf.at[pl.ds(o1, DCH)]], m2,
                              add=True)
        d3 = pltpu.async_copy(obuf, deg_d.at[dbuf.at[pl.ds(o1, DCH)]], m3,
                              add=True)
        d0.wait()
        d1.wait()
        d2.wait()
        d3.wait()
        return carry

    lax.fori_loop(0, DSTEPS // 2, step, 0)
    plsc.subcore_barrier()

    pltpu.sync_copy(deg_s.at[pl.ds(r0, ROWS_PER_TILE)], zbuf)
    pltpu.sync_copy(zbuf, out_hbm.at[c, 0, pl.ds(r0, ROWS_PER_TILE)])
    pltpu.sync_copy(deg_d.at[pl.ds(r0, ROWS_PER_TILE)], zbuf)
    pltpu.sync_copy(zbuf, out_hbm.at[c, 1, pl.ds(r0, ROWS_PER_TILE)])


@functools.partial(
    pl.kernel,
    out_type=jax.ShapeDtypeStruct((2, NPAD, D), jnp.float32),
    mesh=_SC_MESH,
    scratch_types=[
        pltpu.VMEM_SHARED((NPAD, D), jnp.float32),
        pltpu.VMEM((PER_W,), jnp.int32),
        pltpu.VMEM((PER_W,), jnp.int32),
        [pltpu.VMEM((CHUNK, D), jnp.float32) for _ in range(NBUF)],
        [pltpu.SemaphoreType.DMA for _ in range(NBUF)],
        [pltpu.SemaphoreType.DMA for _ in range(NBUF)],
    ],
    compiler_params=pltpu.CompilerParams(use_tc_tiling_on_sc=False),
)
def _sc_aggregate(feat_hbm, src_hbm, dst_hbm, zeros_hbm, out_hbm,
                  agg, sbuf, dbuf, rows, gs, ss):
    c = lax.axis_index("c")
    s = lax.axis_index("s")
    wid = c * NS + s

    base = wid * PER_W
    pltpu.sync_copy(src_hbm.at[pl.ds(base, PER_W)], sbuf)
    pltpu.sync_copy(dst_hbm.at[pl.ds(base, PER_W)], dbuf)
    pltpu.sync_copy(zeros_hbm, rows[0])
    for t in range(ROWS_PER_TILE // ZCH):
        pltpu.sync_copy(rows[0],
                        agg.at[pl.ds(s * ROWS_PER_TILE + t * ZCH, ZCH)])
    plsc.subcore_barrier()

    for b in range(NBUF):
        pltpu.async_copy(feat_hbm.at[sbuf.at[pl.ds(b * CHUNK, CHUNK)]],
                         rows[b], gs[b])

    def step(k, carry):
        j = k * NBUF * CHUNK
        for b in range(NBUF):
            jb = j + b * CHUNK
            didx = dbuf.at[pl.ds(jb, CHUNK)]
            pltpu.make_async_copy(
                feat_hbm.at[sbuf.at[pl.ds(jb, CHUNK)]], rows[b],
                gs[b]).wait()
            pltpu.async_copy(rows[b], agg.at[didx], ss[b], add=True)
        for b in range(NBUF):
            jb = j + b * CHUNK
            didx = dbuf.at[pl.ds(jb, CHUNK)]
            pltpu.make_async_copy(rows[b], agg.at[didx], ss[b]).wait()
            nxt = jb + NBUF * CHUNK

            @pl.when(nxt < PER_W)
            def _():
                pltpu.async_copy(feat_hbm.at[sbuf.at[pl.ds(nxt, CHUNK)]],
                                 rows[b], gs[b])

        return carry

    lax.fori_loop(0, NSTEPS // NBUF, step, 0)
    plsc.subcore_barrier()

    for t in range(ROWS_PER_TILE // ZCH):
        r0 = s * ROWS_PER_TILE + t * ZCH
        pltpu.sync_copy(agg.at[pl.ds(r0, ZCH)], rows[0])
        pltpu.sync_copy(rows[0], out_hbm.at[c, pl.ds(r0, ZCH)])


BLK = 1000
EPS = 1e-5


def _norm_from_deg(degp_blk, which):
    deg = degp_blk[0, which, :, 0] + degp_blk[1, which, :, 0]
    return lax.rsqrt(jnp.maximum(deg, 1.0))


def _embed_body(nf, w, b, degp, x_out, feat_out):
    x = jnp.dot(nf[...], w[...], preferred_element_type=jnp.float32) + b[...]
    x_out[...] = x
    ns = _norm_from_deg(degp[...], 0)
    feat_out[...] = x * ns[:, None]


def _tc_embed(nf, w_emb, b_emb, degp):
    return pl.pallas_call(
        _embed_body,
        grid=(N // BLK,),
        in_specs=[
            pl.BlockSpec((BLK, D), lambda i: (i, 0)),
            pl.BlockSpec((D, D), lambda i: (0, 0)),
            pl.BlockSpec((1, D), lambda i: (0, 0)),
            pl.BlockSpec((2, 2, BLK, 16), lambda i: (0, 0, i, 0)),
        ],
        out_specs=[
            pl.BlockSpec((BLK, D), lambda i: (i, 0)),
            pl.BlockSpec((BLK, D), lambda i: (i, 0)),
        ],
        out_shape=[
            jax.ShapeDtypeStruct((N, D), jnp.float32),
            jax.ShapeDtypeStruct((N, D), jnp.float32),
        ],
    )(nf, w_emb, b_emb, degp)


NB = N // BLK


def _bn_relu_from_acc(h, acc, g, be):
    mean = acc[0, :] / N
    var = acc[1, :] / N - mean * mean
    inv = lax.rsqrt(var + EPS)
    hn = (h - mean) * inv * g + be
    return jnp.maximum(hn, 0.0)


def _mid_body(aggp, x, degp, w1, b1, g, be, w2, b2, feat_out, acc, hbuf):
    i = pl.program_id(0)

    @pl.when(i == 0)
    def _():
        acc[...] = jnp.zeros_like(acc)

    @pl.when(i < NB)
    def _():
        nd = _norm_from_deg(degp[...], 1)
        a = (aggp[0] + aggp[1]) * nd[:, None]
        h = (x[...] + jnp.dot(a, w1[...], preferred_element_type=jnp.float32)
             + b1[...])
        hbuf[pl.ds(i * BLK, BLK), :] = h
        acc[0, :] += jnp.sum(h, axis=0)
        acc[1, :] += jnp.sum(h * h, axis=0)

    @pl.when(i >= NB)
    def _():
        h = hbuf[pl.ds((i - NB) * BLK, BLK), :]
        r = _bn_relu_from_acc(h, acc[...], g[...], be[...])
        tt = jnp.dot(r, w2[...], preferred_element_type=jnp.float32) + b2[...]
        ns = _norm_from_deg(degp[...], 0)
        feat_out[...] = tt * ns[:, None]


def _tc_mid(aggp, x, degp, w1, b1, g, be, w2, b2):
    cst = lambda i: (0, 0)
    p1 = lambda i: (jnp.where(i < NB, i, NB - 1), 0)
    p1agg = lambda i: (0, jnp.where(i < NB, i, NB - 1), 0)
    both = lambda i: (0, 0, i % NB, 0)
    return pl.pallas_call(
        _mid_body,
        grid=(2 * NB,),
        in_specs=[
            pl.BlockSpec((2, BLK, D), p1agg),
            pl.BlockSpec((BLK, D), p1),
            pl.BlockSpec((2, 2, BLK, 16), both),
            pl.BlockSpec((D, D), cst),
            pl.BlockSpec((1, D), cst),
            pl.BlockSpec((1, D), cst),
            pl.BlockSpec((1, D), cst),
            pl.BlockSpec((D, D), cst),
            pl.BlockSpec((1, D), cst),
        ],
        out_specs=pl.BlockSpec((BLK, D), lambda i: (jnp.where(i < NB, 0,
                                                              i - NB), 0)),
        out_shape=jax.ShapeDtypeStruct((N, D), jnp.float32),
        scratch_shapes=[pltpu.VMEM((2, D), jnp.float32),
                        pltpu.VMEM((N, D), jnp.float32)],
    )(aggp, x, degp, w1, b1, g, be, w2, b2)


def _fin_body(aggp, x, degp, w1, b1, g, be, w3, b3, wc, bc,
              hf_out, label_out, acc, hbuf):
    i = pl.program_id(0)

    @pl.when(i == 0)
    def _():
        acc[...] = jnp.zeros_like(acc)

    @pl.when(i < NB)
    def _():
        nd = _norm_from_deg(degp[...], 1)
        a = (aggp[0] + aggp[1]) * nd[:, None]
        h = (x[...] + jnp.dot(a, w1[...], preferred_element_type=jnp.float32)
             + b1[...])
        hbuf[pl.ds(i * BLK, BLK), :] = h
        acc[0, :] += jnp.sum(h, axis=0)
        acc[1, :] += jnp.sum(h * h, axis=0)

    @pl.when(i >= NB)
    def _():
        h = hbuf[pl.ds((i - NB) * BLK, BLK), :]
        r = _bn_relu_from_acc(h, acc[...], g[...], be[...])
        ho = jnp.dot(r, w3[...], preferred_element_type=jnp.float32) + b3[...]
        hf_out[...] = ho
        acc[2, :] += jnp.sum(ho, axis=0)

    @pl.when(i == 2 * NB - 1)
    def _():
        y = acc[2:3, :] / N
        label_out[...] = (
            jnp.dot(y, wc[...], preferred_element_type=jnp.float32) + bc[...]
        )


def _tc_fin(aggp, x, degp, w1, b1, g, be, w3, b3, wc, bc):
    cst = lambda i: (0, 0)
    p1 = lambda i: (jnp.where(i < NB, i, NB - 1), 0)
    p1agg = lambda i: (0, jnp.where(i < NB, i, NB - 1), 0)
    both = lambda i: (0, 0, i % NB, 0)
    return pl.pallas_call(
        _fin_body,
        grid=(2 * NB,),
        in_specs=[
            pl.BlockSpec((2, BLK, D), p1agg),
            pl.BlockSpec((BLK, D), p1),
            pl.BlockSpec((2, 2, BLK, 16), both),
            pl.BlockSpec((D, D), cst),
            pl.BlockSpec((1, D), cst),
            pl.BlockSpec((1, D), cst),
            pl.BlockSpec((1, D), cst),
            pl.BlockSpec((D, D), cst),
            pl.BlockSpec((1, D), cst),
            pl.BlockSpec((D, C), cst),
            pl.BlockSpec((1, C), cst),
        ],
        out_specs=[
            pl.BlockSpec((BLK, D), lambda i: (jnp.where(i < NB, 0, i - NB),
                                              0)),
            pl.BlockSpec((1, C), cst),
        ],
        out_shape=[
            jax.ShapeDtypeStruct((N, D), jnp.float32),
            jax.ShapeDtypeStruct((1, C), jnp.float32),
        ],
        scratch_shapes=[pltpu.VMEM((3, D), jnp.float32),
                        pltpu.VMEM((N, D), jnp.float32)],
    )(aggp, x, degp, w1, b1, g, be, w3, b3, wc, bc)


def kernel(node_features, edge_index, W_emb, b_emb, W_c1, b_c1, g1, be1,
           W_l2, b_l2, W_c2, b_c2, g2, be2, W_l3, b_l3, W_cls, b_cls):
    src1 = edge_index[0]
    dst1 = edge_index[1]
    ones16 = jnp.ones((DCH, 16), jnp.float32)
    zeros16 = jnp.zeros((ROWS_PER_TILE, 16), jnp.float32)
    zeros128 = jnp.zeros((ZCH, D), jnp.float32)

    degp = _sc_degrees(src1, dst1, ones16, zeros16)
    x, feat1 = _tc_embed(node_features, W_emb, b_emb.reshape(1, D), degp)
    agg1 = _sc_aggregate(feat1, src1, dst1, zeros128)
    feat2 = _tc_mid(agg1, x, degp, W_c1, b_c1.reshape(1, D),
                    g1.reshape(1, D), be1.reshape(1, D),
                    W_l2, b_l2.reshape(1, D))
    agg2 = _sc_aggregate(feat2, src1, dst1, zeros128)
    h_out, label = _tc_fin(agg2, x, degp, W_c2, b_c2.reshape(1, D),
                           g2.reshape(1, D), be2.reshape(1, D),
                           W_l3, b_l3.reshape(1, D), W_cls,
                           b_cls.reshape(1, C))
    return (h_out, label)


# speedup vs baseline: 11.8787x; 1.0923x over previous
import functools

import jax
import jax.numpy as jnp
from jax import lax
from jax.experimental import pallas as pl
from jax.experimental.pallas import tpu as pltpu
from jax.experimental.pallas import tpu_sc as plsc

N = 10000
E = 320000
D = 128
C = 60

NC = 2
NS = 16
NW = NC * NS
PER_W = E // NW
DCH = 200
DSTEPS = PER_W // DCH
CHUNK = 40
NSTEPS = PER_W // CHUNK
NBUF = 5
NPAD = 10240
ZCH = 40
ROWS_PER_TILE = NPAD // NS

_SC_MESH = plsc.VectorSubcoreMesh(core_axis_name="c", subcore_axis_name="s")


@functools.partial(
    pl.kernel,
    out_type=jax.ShapeDtypeStruct((2, 2, NPAD, 16), jnp.float32),
    mesh=_SC_MESH,
    scratch_types=[
        pltpu.VMEM_SHARED((NPAD, 16), jnp.float32),
        pltpu.VMEM_SHARED((NPAD, 16), jnp.float32),
        pltpu.VMEM((PER_W,), jnp.int32),
        pltpu.VMEM((PER_W,), jnp.int32),
        pltpu.VMEM((DCH, 16), jnp.float32),
        pltpu.VMEM((ROWS_PER_TILE, 16), jnp.float32),
        pltpu.SemaphoreType.DMA,
        pltpu.SemaphoreType.DMA,
        pltpu.SemaphoreType.DMA,
        pltpu.SemaphoreType.DMA,
    ],
    compiler_params=pltpu.CompilerParams(use_tc_tiling_on_sc=False),
)
def _sc_degrees(edge_hbm, ones_hbm, zeros_hbm, out_hbm,
                deg_s, deg_d, sbuf, dbuf, obuf, zbuf, m0, m1, m2, m3):
    c = lax.axis_index("c")
    s = lax.axis_index("s")
    wid = c * NS + s

    pltpu.sync_copy(zeros_hbm, zbuf)
    r0 = s * ROWS_PER_TILE
    pltpu.sync_copy(zbuf, deg_s.at[pl.ds(r0, ROWS_PER_TILE)])
    pltpu.sync_copy(zbuf, deg_d.at[pl.ds(r0, ROWS_PER_TILE)])
    pltpu.sync_copy(ones_hbm, obuf)
    base = wid * PER_W
    pltpu.sync_copy(edge_hbm.at[0, pl.ds(base, PER_W)], sbuf)
    pltpu.sync_copy(edge_hbm.at[1, pl.ds(base, PER_W)], dbuf)
    plsc.subcore_barrier()

    def step(k, carry):
        o0 = k * 2 * DCH
        o1 = o0 + DCH
        d0 = pltpu.async_copy(obuf, deg_s.at[sbuf.at[pl.ds(o0, DCH)]], m0,
                              add=True)
        d1 = pltpu.async_copy(obuf, deg_d.at[dbuf.at[pl.ds(o0, DCH)]], m1,
                              add=True)
        d2 = pltpu.async_copy(obuf, deg_s.at[sbuf.at[pl.ds(o1, DCH)]], m2,
                              add=True)
        d3 = pltpu.async_copy(obuf, deg_d.at[dbuf.at[pl.ds(o1, DCH)]], m3,
                              add=True)
        d0.wait()
        d1.wait()
        d2.wait()
        d3.wait()
        return carry

    lax.fori_loop(0, DSTEPS // 2, step, 0)
    plsc.subcore_barrier()

    pltpu.sync_copy(deg_s.at[pl.ds(r0, ROWS_PER_TILE)], zbuf)
    pltpu.sync_copy(zbuf, out_hbm.at[c, 0, pl.ds(r0, ROWS_PER_TILE)])
    pltpu.sync_copy(deg_d.at[pl.ds(r0, ROWS_PER_TILE)], zbuf)
    pltpu.sync_copy(zbuf, out_hbm.at[c, 1, pl.ds(r0, ROWS_PER_TILE)])


@functools.partial(
    pl.kernel,
    out_type=jax.ShapeDtypeStruct((2, NPAD, D), jnp.float32),
    mesh=_SC_MESH,
    scratch_types=[
        pltpu.VMEM_SHARED((NPAD, D), jnp.float32),
        pltpu.VMEM((PER_W,), jnp.int32),
        pltpu.VMEM((PER_W,), jnp.int32),
        [pltpu.VMEM((CHUNK, D), jnp.float32) for _ in range(NBUF)],
        [pltpu.SemaphoreType.DMA for _ in range(NBUF)],
        [pltpu.SemaphoreType.DMA for _ in range(NBUF)],
    ],
    compiler_params=pltpu.CompilerParams(use_tc_tiling_on_sc=False),
)
def _sc_aggregate(feat_hbm, edge_hbm, zeros_hbm, out_hbm,
                  agg, sbuf, dbuf, rows, gs, ss):
    c = lax.axis_index("c")
    s = lax.axis_index("s")
    wid = c * NS + s

    base = wid * PER_W
    pltpu.sync_copy(edge_hbm.at[0, pl.ds(base, PER_W)], sbuf)
    pltpu.sync_copy(edge_hbm.at[1, pl.ds(base, PER_W)], dbuf)
    pltpu.sync_copy(zeros_hbm, rows[0])
    zd = []
    for t in range(ROWS_PER_TILE // ZCH):
        zd.append(pltpu.async_copy(
            rows[0], agg.at[pl.ds(s * ROWS_PER_TILE + t * ZCH, ZCH)],
            ss[t % 2]))
    for d in zd:
        d.wait()
    plsc.subcore_barrier()

    for b in range(NBUF):
        pltpu.async_copy(feat_hbm.at[sbuf.at[pl.ds(b * CHUNK, CHUNK)]],
                         rows[b], gs[b])

    def step(k, carry):
        j = k * NBUF * CHUNK
        for b in range(NBUF):
            jb = j + b * CHUNK
            didx = dbuf.at[pl.ds(jb, CHUNK)]
            pltpu.make_async_copy(
                feat_hbm.at[sbuf.at[pl.ds(jb, CHUNK)]], rows[b],
                gs[b]).wait()
            pltpu.async_copy(rows[b], agg.at[didx], ss[b], add=True)
        for b in range(NBUF):
            jb = j + b * CHUNK
            didx = dbuf.at[pl.ds(jb, CHUNK)]
            pltpu.make_async_copy(rows[b], agg.at[didx], ss[b]).wait()
            nxt = jb + NBUF * CHUNK

            @pl.when(nxt < PER_W)
            def _():
                pltpu.async_copy(feat_hbm.at[sbuf.at[pl.ds(nxt, CHUNK)]],
                                 rows[b], gs[b])

        return carry

    lax.fori_loop(0, NSTEPS // NBUF, step, 0)
    plsc.subcore_barrier()

    nexp = ROWS_PER_TILE // ZCH
    ind = [None] * nexp
    outd = [None] * nexp
    for t in range(NBUF):
        r0 = s * ROWS_PER_TILE + t * ZCH
        ind[t] = pltpu.async_copy(agg.at[pl.ds(r0, ZCH)], rows[t % NBUF],
                                  gs[t % NBUF])
    for t in range(nexp):
        b = t % NBUF
        r0 = s * ROWS_PER_TILE + t * ZCH
        ind[t].wait()
        outd[t] = pltpu.async_copy(rows[b], out_hbm.at[c, pl.ds(r0, ZCH)],
                                   ss[b])
        nt = t + NBUF
        if nt < nexp:
            outd[t].wait()
            rn = s * ROWS_PER_TILE + nt * ZCH
            ind[nt] = pltpu.async_copy(agg.at[pl.ds(rn, ZCH)], rows[b],
                                       gs[b])
    for t in range(nexp - NBUF, nexp):
        outd[t].wait()


BLK = 2000
EPS = 1e-5


def _norm_from_deg(degp_blk, which):
    deg = degp_blk[0, which, :, 0] + degp_blk[1, which, :, 0]
    return lax.rsqrt(jnp.maximum(deg, 1.0))


def _embed_body(nf, w, b, degp, x_out, feat_out):
    x = jnp.dot(nf[...], w[...], preferred_element_type=jnp.float32) + b[...]
    x_out[...] = x
    ns = _norm_from_deg(degp[...], 0)
    feat_out[...] = x * ns[:, None]


def _tc_embed(nf, w_emb, b_emb, degp):
    return pl.pallas_call(
        _embed_body,
        grid=(N // BLK,),
        in_specs=[
            pl.BlockSpec((BLK, D), lambda i: (i, 0)),
            pl.BlockSpec((D, D), lambda i: (0, 0)),
            pl.BlockSpec((1, D), lambda i: (0, 0)),
            pl.BlockSpec((2, 2, BLK, 16), lambda i: (0, 0, i, 0)),
        ],
        out_specs=[
            pl.BlockSpec((BLK, D), lambda i: (i, 0)),
            pl.BlockSpec((BLK, D), lambda i: (i, 0)),
        ],
        out_shape=[
            jax.ShapeDtypeStruct((N, D), jnp.float32),
            jax.ShapeDtypeStruct((N, D), jnp.float32),
        ],
    )(nf, w_emb, b_emb, degp)


NB = N // BLK


def _bn_relu_from_acc(h, acc, g, be):
    mean = acc[0, :] / N
    var = acc[1, :] / N - mean * mean
    inv = lax.rsqrt(var + EPS)
    hn = (h - mean) * inv * g + be
    return jnp.maximum(hn, 0.0)


def _mid_body(aggp, x, degp, w1, b1, g, be, w2, b2, feat_out, acc, hbuf):
    i = pl.program_id(0)

    @pl.when(i == 0)
    def _():
        acc[...] = jnp.zeros_like(acc)

    @pl.when(i < NB)
    def _():
        nd = _norm_from_deg(degp[...], 1)
        a = (aggp[0] + aggp[1]) * nd[:, None]
        h = (x[...] + jnp.dot(a, w1[...], preferred_element_type=jnp.float32)
             + b1[...])
        hbuf[pl.ds(i * BLK, BLK), :] = h
        acc[0, :] += jnp.sum(h, axis=0)
        acc[1, :] += jnp.sum(h * h, axis=0)

    @pl.when(i >= NB)
    def _():
        h = hbuf[pl.ds((i - NB) * BLK, BLK), :]
        r = _bn_relu_from_acc(h, acc[...], g[...], be[...])
        tt = jnp.dot(r, w2[...], preferred_element_type=jnp.float32) + b2[...]
        ns = _norm_from_deg(degp[...], 0)
        feat_out[...] = tt * ns[:, None]


def _tc_mid(aggp, x, degp, w1, b1, g, be, w2, b2):
    cst = lambda i: (0, 0)
    p1 = lambda i: (jnp.where(i < NB, i, NB - 1), 0)
    p1agg = lambda i: (0, jnp.where(i < NB, i, NB - 1), 0)
    both = lambda i: (0, 0, i % NB, 0)
    return pl.pallas_call(
        _mid_body,
        grid=(2 * NB,),
        in_specs=[
            pl.BlockSpec((2, BLK, D), p1agg),
            pl.BlockSpec((BLK, D), p1),
            pl.BlockSpec((2, 2, BLK, 16), both),
            pl.BlockSpec((D, D), cst),
            pl.BlockSpec((1, D), cst),
            pl.BlockSpec((1, D), cst),
            pl.BlockSpec((1, D), cst),
            pl.BlockSpec((D, D), cst),
            pl.BlockSpec((1, D), cst),
        ],
        out_specs=pl.BlockSpec((BLK, D), lambda i: (jnp.where(i < NB, 0,
                                                              i - NB), 0)),
        out_shape=jax.ShapeDtypeStruct((N, D), jnp.float32),
        scratch_shapes=[pltpu.VMEM((2, D), jnp.float32),
                        pltpu.VMEM((N, D), jnp.float32)],
    )(aggp, x, degp, w1, b1, g, be, w2, b2)


def _fin_body(aggp, x, degp, w1, b1, g, be, w3, b3, wc, bc,
              hf_out, label_out, acc, hbuf):
    i = pl.program_id(0)

    @pl.when(i == 0)
    def _():
        acc[...] = jnp.zeros_like(acc)

    @pl.when(i < NB)
    def _():
        nd = _norm_from_deg(degp[...], 1)
        a = (aggp[0] + aggp[1]) * nd[:, None]
        h = (x[...] + jnp.dot(a, w1[...], preferred_element_type=jnp.float32)
             + b1[...])
        hbuf[pl.ds(i * BLK, BLK), :] = h
        acc[0, :] += jnp.sum(h, axis=0)
        acc[1, :] += jnp.sum(h * h, axis=0)

    @pl.when(i >= NB)
    def _():
        h = hbuf[pl.ds((i - NB) * BLK, BLK), :]
        r = _bn_relu_from_acc(h, acc[...], g[...], be[...])
        ho = jnp.dot(r, w3[...], preferred_element_type=jnp.float32) + b3[...]
        hf_out[...] = ho
        acc[2, :] += jnp.sum(ho, axis=0)

    @pl.when(i == 2 * NB - 1)
    def _():
        y = acc[2:3, :] / N
        label_out[...] = (
            jnp.dot(y, wc[...], preferred_element_type=jnp.float32) + bc[...]
        )


def _tc_fin(aggp, x, degp, w1, b1, g, be, w3, b3, wc, bc):
    cst = lambda i: (0, 0)
    p1 = lambda i: (jnp.where(i < NB, i, NB - 1), 0)
    p1agg = lambda i: (0, jnp.where(i < NB, i, NB - 1), 0)
    both = lambda i: (0, 0, i % NB, 0)
    return pl.pallas_call(
        _fin_body,
        grid=(2 * NB,),
        in_specs=[
            pl.BlockSpec((2, BLK, D), p1agg),
            pl.BlockSpec((BLK, D), p1),
            pl.BlockSpec((2, 2, BLK, 16), both),
            pl.BlockSpec((D, D), cst),
            pl.BlockSpec((1, D), cst),
            pl.BlockSpec((1, D), cst),
            pl.BlockSpec((1, D), cst),
            pl.BlockSpec((D, D), cst),
            pl.BlockSpec((1, D), cst),
            pl.BlockSpec((D, C), cst),
            pl.BlockSpec((1, C), cst),
        ],
        out_specs=[
            pl.BlockSpec((BLK, D), lambda i: (jnp.where(i < NB, 0, i - NB),
                                              0)),
            pl.BlockSpec((1, C), cst),
        ],
        out_shape=[
            jax.ShapeDtypeStruct((N, D), jnp.float32),
            jax.ShapeDtypeStruct((1, C), jnp.float32),
        ],
        scratch_shapes=[pltpu.VMEM((3, D), jnp.float32),
                        pltpu.VMEM((N, D), jnp.float32)],
    )(aggp, x, degp, w1, b1, g, be, w3, b3, wc, bc)


def kernel(node_features, edge_index, W_emb, b_emb, W_c1, b_c1, g1, be1,
           W_l2, b_l2, W_c2, b_c2, g2, be2, W_l3, b_l3, W_cls, b_cls):
    ones16 = jnp.ones((DCH, 16), jnp.float32)
    zeros16 = jnp.zeros((ROWS_PER_TILE, 16), jnp.float32)
    zeros128 = jnp.zeros((ZCH, D), jnp.float32)

    degp = _sc_degrees(edge_index, ones16, zeros16)
    x, feat1 = _tc_embed(node_features, W_emb, b_emb.reshape(1, D), degp)
    agg1 = _sc_aggregate(feat1, edge_index, zeros128)
    feat2 = _tc_mid(agg1, x, degp, W_c1, b_c1.reshape(1, D),
                    g1.reshape(1, D), be1.reshape(1, D),
                    W_l2, b_l2.reshape(1, D))
    agg2 = _sc_aggregate(feat2, edge_index, zeros128)
    h_out, label = _tc_fin(agg2, x, degp, W_c2, b_c2.reshape(1, D),
                           g2.reshape(1, D), be2.reshape(1, D),
                           W_l3, b_l3.reshape(1, D), W_cls,
                           b_cls.reshape(1, C))
    return (h_out, label)
